# Initial kernel scaffold; baseline (speedup 1.0000x reference)
import jax
import jax.numpy as jnp
from jax.experimental import pallas as pl


def kernel(x_seq, edge_index, node_idx, apart_feature, W1a, b1a, g1a, be1a, W1b, b1b, W2a, b2a, g2a, be2a, W2b, b2b, Wih, Whh, bih, bhh, Wfc1, bfc1, Wfc2, bfc2, Wfc3, bfc3):
    raise NotImplementedError("write your pallas kernel here")


# speedup vs baseline: 15.0347x; 15.0347x over previous
import functools
import math

import jax
import jax.numpy as jnp
from jax import lax
from jax.experimental import pallas as pl
from jax.experimental.pallas import tpu as pltpu
from jax.experimental.pallas import tpu_sc as plsc

KB = 128
NSC = 2
NSUB = 16
NW = NSC * NSUB


def _cdiv(a, b):
    return (a + b - 1) // b


def _prep_edges(src, dst, n_nodes, chunk, nchunk):
    e = src.shape[0]
    ep_cap = e + nchunk * KB
    perm = jnp.argsort(dst)
    sdst = dst[perm]
    ssrc = src[perm]
    chunk_of = jnp.minimum(sdst // chunk, nchunk - 1)
    edges = jnp.arange(1, nchunk + 1, dtype=jnp.int32) * chunk
    bounds = jnp.searchsorted(sdst, edges, side="left").astype(jnp.int32)
    bounds = bounds.at[-1].set(e)
    starts = jnp.concatenate([jnp.zeros((1,), jnp.int32), bounds[:-1]])
    cnt = bounds - starts
    nb = _cdiv_arr(cnt)
    pstart = jnp.concatenate([jnp.zeros((1,), jnp.int32),
                              jnp.cumsum(nb).astype(jnp.int32)])
    local = jnp.arange(e, dtype=jnp.int32) - starts[chunk_of]
    pos = (pstart[chunk_of] * KB + local).astype(jnp.int32)
    ar = jnp.arange(ep_cap, dtype=jnp.int32)
    pad_src = (ar * 997) % n_nodes
    pad_dst = chunk + (ar % NSUB)
    psrc = pad_src.at[pos].set(ssrc)
    pldst = pad_dst.at[pos].set(sdst - chunk_of * chunk)
    ctrl = (jnp.zeros((nchunk, 16), jnp.int32)
            .at[:, 0].set(pstart[:-1]).at[:, 1].set(nb).reshape(-1))
    return psrc, pldst, ctrl


def _cdiv_arr(x):
    return (x + (KB - 1)) // KB


def _sc_aggregate(x_rows, psrc, pldst, ctrl, chunk, nchunk):
    n, f = x_rows.shape
    g_cnt = f // 128
    npad = nchunk * chunk
    acc_rows = chunk + 128
    nzr = acc_rows // NSUB
    nwr = chunk // NSUB
    ctrl_len = ctrl.shape[0]
    mesh = plsc.VectorSubcoreMesh(core_axis_name="c", subcore_axis_name="s")

    @functools.partial(
        pl.kernel, mesh=mesh,
        out_type=jax.ShapeDtypeStruct((npad, f), jnp.float32),
        scratch_types=[
            pltpu.VMEM((KB,), jnp.int32),
            pltpu.VMEM((KB,), jnp.int32),
            pltpu.VMEM((KB, 128), jnp.float32),
            pltpu.VMEM((NSUB, 128), jnp.float32),
            pltpu.VMEM((ctrl_len,), jnp.int32),
            pltpu.VMEM_SHARED((acc_rows, 128), jnp.float32),
            pltpu.SemaphoreType.DMA,
        ],
    )
    def agg_kernel(x_hbm, psrc_hbm, pldst_hbm, ctrl_hbm, out_hbm,
                   sidx_v, didx_v, rows_v, zero_v, ctrl_v, acc_sh, sem):
        c = lax.axis_index("c")
        s = lax.axis_index("s")
        pltpu.sync_copy(ctrl_hbm, ctrl_v)

        def zinit(i, _):
            zero_v[i // 8, pl.ds((i % 8) * 16, 16)] = jnp.zeros((16,),
                                                                jnp.float32)
            return 0
        lax.fori_loop(0, NSUB * 8, zinit, 0)

        for g in range(g_cnt):
            def chunk_body(ci, _):
                chunk_id = ci * NSC + c
                rec = ctrl_v[pl.ds(chunk_id * 16, 16)]
                base_batch = rec[0]
                nb = rec[1]
                z0 = s * nzr
                for k in range(nzr // NSUB):
                    pltpu.sync_copy(zero_v,
                                    acc_sh.at[pl.ds(z0 + k * NSUB, NSUB)])
                rem = nzr % NSUB
                if rem:
                    pltpu.sync_copy(
                        zero_v.at[pl.ds(0, rem)],
                        acc_sh.at[pl.ds(z0 + (nzr // NSUB) * NSUB, rem)])
                plsc.subcore_barrier()
                nb_s = jnp.maximum(0, (nb - s + NSUB - 1) // NSUB)

                def batch_body(j, _):
                    e0 = (base_batch + s + j * NSUB) * KB
                    pltpu.sync_copy(psrc_hbm.at[pl.ds(e0, KB)], sidx_v)
                    pltpu.sync_copy(pldst_hbm.at[pl.ds(e0, KB)], didx_v)
                    if g_cnt == 1:
                        pltpu.async_copy(x_hbm.at[sidx_v], rows_v, sem).wait()
                    else:
                        pltpu.async_copy(
                            x_hbm.at[sidx_v, pl.ds(g * 128, 128)],
                            rows_v, sem).wait()
                    pltpu.sync_copy(rows_v, acc_sh.at[didx_v], add=True)
                    return 0
                lax.fori_loop(0, nb_s, batch_body, 0)
                plsc.subcore_barrier()
                w0 = s * nwr
                if g_cnt == 1:
                    pltpu.sync_copy(
                        acc_sh.at[pl.ds(w0, nwr)],
                        out_hbm.at[pl.ds(chunk_id * chunk + w0, nwr)])
                else:
                    pltpu.sync_copy(
                        acc_sh.at[pl.ds(w0, nwr)],
                        out_hbm.at[pl.ds(chunk_id * chunk + w0, nwr),
                                   pl.ds(g * 128, 128)])
                plsc.subcore_barrier()
                return 0
            lax.fori_loop(0, nchunk // NSC, chunk_body, 0)

    return agg_kernel(x_rows, psrc, pldst, ctrl)


def _sc_gather(table, idx):
    n, f = table.shape
    b = idx.shape[0]
    per_w = b // (NSC * NSUB)
    mesh = plsc.VectorSubcoreMesh(core_axis_name="c", subcore_axis_name="s")

    @functools.partial(
        pl.kernel, mesh=mesh,
        out_type=jax.ShapeDtypeStruct((b, f), jnp.float32),
        scratch_types=[
            pltpu.VMEM((per_w,), jnp.int32),
            pltpu.VMEM((per_w, f), jnp.float32),
            pltpu.SemaphoreType.DMA,
        ],
    )
    def gather_kernel(tab_hbm, idx_hbm, out_hbm, idx_v, rows_v, sem):
        wid = lax.axis_index("s") * NSC + lax.axis_index("c")
        base = wid * per_w
        pltpu.sync_copy(idx_hbm.at[pl.ds(base, per_w)], idx_v)
        pltpu.async_copy(tab_hbm.at[idx_v], rows_v, sem).wait()
        pltpu.sync_copy(rows_v, out_hbm.at[pl.ds(base, per_w)])

    return gather_kernel(table, idx)


def _tc_stats(x, agg, wbd, bbd, bn, t):
    n, fin = x.shape
    fout = wbd.shape[1]
    grid = n // bn

    def body(xr, ar, wr, br, sums, sqs):
        i = pl.program_id(0)
        h = jnp.maximum(jnp.dot(xr[...] + ar[...], wr[...],
                                preferred_element_type=jnp.float32, precision=jax.lax.Precision.HIGHEST)
                        + br[...], 0.0)

        @pl.when(i == 0)
        def _():
            sums[...] = jnp.zeros((8, fout), jnp.float32)
            sqs[...] = jnp.zeros((8, fout), jnp.float32)
        r = i % 8
        part = jnp.sum(h, axis=0, keepdims=True)
        part2 = jnp.sum(h * h, axis=0, keepdims=True)
        sums[pl.ds(r, 1), :] += part
        sqs[pl.ds(r, 1), :] += part2

    return pl.pallas_call(
        body,
        grid=(grid,),
        in_specs=[
            pl.BlockSpec((bn, fin), lambda i: (i, 0)),
            pl.BlockSpec((bn, fin), lambda i: (i, 0)),
            pl.BlockSpec((fin, fout), lambda i: (0, 0)),
            pl.BlockSpec((1, fout), lambda i: (0, 0)),
        ],
        out_specs=[
            pl.BlockSpec((8, fout), lambda i: (0, 0)),
            pl.BlockSpec((8, fout), lambda i: (0, 0)),
        ],
        out_shape=[
            jax.ShapeDtypeStruct((8, fout), jnp.float32),
            jax.ShapeDtypeStruct((8, fout), jnp.float32),
        ],
    )(x, agg, wbd, bbd)


def _tc_main(x, agg, sums, sqs, wbd, bbd, g_t, be_t, wb, bb,
             wih_t, whh_t, bih, bhh, bn, t, h, emit_all):
    n, fin = x.shape
    fout = wbd.shape[1]
    n_total = float(n)
    grid = n // bn
    out_w = fout if emit_all else h

    def body(xr, ar, sums_r, sqs_r, wr, br, gr, ber, wbr, bbr,
             wihr, whhr, bihr, bhhr, out):
        hpre = jnp.maximum(jnp.dot(xr[...] + ar[...], wr[...],
                                   preferred_element_type=jnp.float32, precision=jax.lax.Precision.HIGHEST)
                           + br[...], 0.0)
        tot = jnp.sum(sums_r[...], axis=0, keepdims=True)
        tot2 = jnp.sum(sqs_r[...], axis=0, keepdims=True)
        mean = tot / n_total
        var = tot2 / n_total - mean * mean
        rstd = lax.rsqrt(var + 1e-5)
        scale = rstd * gr[...]
        shift = ber[...] - mean * scale
        hn = hpre * scale + shift
        carry = jnp.zeros((bn, h), jnp.float32)
        for ti in range(t):
            ht = hn[:, ti * h:(ti + 1) * h]
            h1 = jnp.maximum(jnp.dot(ht, wbr[...],
                                     preferred_element_type=jnp.float32, precision=jax.lax.Precision.HIGHEST)
                             + bbr[...], 0.0)
            gi = jnp.dot(h1, wihr[...],
                         preferred_element_type=jnp.float32, precision=jax.lax.Precision.HIGHEST) + bihr[...]
            gh = jnp.dot(carry, whhr[...],
                         preferred_element_type=jnp.float32, precision=jax.lax.Precision.HIGHEST) + bhhr[...]
            r = jax.nn.sigmoid(gi[:, 0:h] + gh[:, 0:h])
            z = jax.nn.sigmoid(gi[:, h:2 * h] + gh[:, h:2 * h])
            nn = jnp.tanh(gi[:, 2 * h:3 * h] + r * gh[:, 2 * h:3 * h])
            carry = (1.0 - z) * nn + z * carry
            if emit_all:
                out[:, ti * h:(ti + 1) * h] = carry
        if not emit_all:
            out[...] = carry

    return pl.pallas_call(
        body,
        grid=(grid,),
        in_specs=[
            pl.BlockSpec((bn, fin), lambda i: (i, 0)),
            pl.BlockSpec((bn, fin), lambda i: (i, 0)),
            pl.BlockSpec((8, fout), lambda i: (0, 0)),
            pl.BlockSpec((8, fout), lambda i: (0, 0)),
            pl.BlockSpec((fin, fout), lambda i: (0, 0)),
            pl.BlockSpec((1, fout), lambda i: (0, 0)),
            pl.BlockSpec((1, fout), lambda i: (0, 0)),
            pl.BlockSpec((1, fout), lambda i: (0, 0)),
            pl.BlockSpec((h, h), lambda i: (0, 0)),
            pl.BlockSpec((1, h), lambda i: (0, 0)),
            pl.BlockSpec((h, 3 * h), lambda i: (0, 0)),
            pl.BlockSpec((h, 3 * h), lambda i: (0, 0)),
            pl.BlockSpec((1, 3 * h), lambda i: (0, 0)),
            pl.BlockSpec((1, 3 * h), lambda i: (0, 0)),
        ],
        out_specs=pl.BlockSpec((bn, out_w), lambda i: (i, 0)),
        out_shape=jax.ShapeDtypeStruct((n, out_w), jnp.float32),
    )(x, agg, sums, sqs, wbd, bbd, g_t, be_t, wb, bb,
      wih_t, whh_t, bih, bhh)


def _tc_head(hg, apt, wf1a, wf1b, bfc1, wfc2, bfc2, wfc3, bfc3):
    b, h = hg.shape
    h2 = wfc2.shape[1]

    def body(hgr, aptr, w1ar, w1br, b1r, w2r, b2r, w3r, b3r, out):
        y = (jnp.dot(hgr[...], w1ar[...], preferred_element_type=jnp.float32, precision=jax.lax.Precision.HIGHEST)
             + jnp.dot(aptr[...], w1br[...], preferred_element_type=jnp.float32, precision=jax.lax.Precision.HIGHEST)
             + b1r[...])
        y = jnp.where(y > 0, y, 0.1 * y)
        y = jnp.dot(y, w2r[...], preferred_element_type=jnp.float32, precision=jax.lax.Precision.HIGHEST) + b2r[...]
        y = jnp.where(y > 0, y, 0.05 * y)
        out[...] = (jnp.dot(y, w3r[...], preferred_element_type=jnp.float32, precision=jax.lax.Precision.HIGHEST)
                    + b3r[...])

    apt_w = apt.shape[1]
    h3 = wfc3.shape[0]
    return pl.pallas_call(
        body,
        out_shape=jax.ShapeDtypeStruct((b, 128), jnp.float32),
    )(hg, apt, wf1a, wf1b, bfc1.reshape(1, -1), wfc2,
      bfc2.reshape(1, -1), jnp.pad(wfc3, ((0, 0), (0, 127))),
      jnp.pad(bfc3, (0, 127)).reshape(1, -1))


def _block_diag(w, t):
    a, b = w.shape
    eye = jnp.eye(t, dtype=w.dtype)
    return (eye[:, None, :, None] * w[None, :, None, :]).reshape(t * a, t * b)


def kernel(x_seq, edge_index, node_idx, apart_feature, W1a, b1a, g1a, be1a,
           W1b, b1b, W2a, b2a, g2a, be2a, W2b, b2b, Wih, Whh, bih, bhh,
           Wfc1, bfc1, Wfc2, bfc2, Wfc3, bfc3):
    t, n, in_dim = x_seq.shape
    h = W1a.shape[1]
    chunk = 4096
    nchunk = _cdiv(n, chunk)
    if nchunk % NSC:
        nchunk += 1
    bn = 1000 if n % 1000 == 0 else n

    src, dst = edge_index[0], edge_index[1]
    psrc, pldst, ctrl = _prep_edges(src, dst, n, chunk, nchunk)

    f1 = t * in_dim
    f1p = _cdiv(f1, 128) * 128
    xp = jnp.pad(jnp.transpose(x_seq, (1, 0, 2)).reshape(n, f1),
                 ((0, 0), (0, f1p - f1)))

    w1a_bd = jnp.pad(_block_diag(W1a, t), ((0, f1p - f1), (0, 0)))
    b1a_t = jnp.tile(b1a, t).reshape(1, -1)
    g1a_t = jnp.tile(g1a, t).reshape(1, -1)
    be1a_t = jnp.tile(be1a, t).reshape(1, -1)
    w2a_bd = _block_diag(W2a, t)
    b2a_t = jnp.tile(b2a, t).reshape(1, -1)
    g2a_t = jnp.tile(g2a, t).reshape(1, -1)
    be2a_t = jnp.tile(be2a, t).reshape(1, -1)
    wih_t = Wih.T
    whh_t = Whh.T
    bih_r = bih.reshape(1, -1)
    bhh_r = bhh.reshape(1, -1)
    b1b_r = b1b.reshape(1, -1)
    b2b_r = b2b.reshape(1, -1)

    agg1 = _sc_aggregate(xp, psrc, pldst, ctrl, chunk, nchunk)[:n]
    s1, q1 = _tc_stats(xp, agg1, w1a_bd, b1a_t, bn, t)
    outs = _tc_main(xp, agg1, s1, q1, w1a_bd, b1a_t, g1a_t, be1a_t,
                    W1b, b1b_r, wih_t, whh_t, bih_r, bhh_r,
                    bn, t, h, emit_all=True)

    agg2 = _sc_aggregate(outs, psrc, pldst, ctrl, chunk, nchunk)[:n]
    s2, q2 = _tc_stats(outs, agg2, w2a_bd, b2a_t, bn, t)
    h_last = _tc_main(outs, agg2, s2, q2, w2a_bd, b2a_t, g2a_t, be2a_t,
                      W2b, b2b_r, wih_t, whh_t, bih_r, bhh_r,
                      bn, t, h, emit_all=False)

    hg = _sc_gather(jnp.pad(h_last, ((0, 0), (0, 128 - h))), node_idx)[:, :h]
    y = _tc_head(hg, apart_feature, Wfc1[:h], Wfc1[h:], bfc1,
                 Wfc2, bfc2, Wfc3, bfc3)
    return y[:, :1]


# speedup vs baseline: 25.9871x; 1.7285x over previous
import functools
import math

import jax
import jax.numpy as jnp
from jax import lax
from jax.experimental import pallas as pl
from jax.experimental.pallas import tpu as pltpu
from jax.experimental.pallas import tpu_sc as plsc

KB = 128
NSC = 2
NSUB = 16
NW = NSC * NSUB


def _cdiv(a, b):
    return (a + b - 1) // b


def _prep_edges(src, dst, n_nodes, chunk, nchunk):
    e = src.shape[0]
    ep_cap = e + nchunk * KB
    perm = jnp.argsort(dst)
    sdst = dst[perm]
    ssrc = src[perm]
    chunk_of = jnp.minimum(sdst // chunk, nchunk - 1)
    edges = jnp.arange(1, nchunk + 1, dtype=jnp.int32) * chunk
    bounds = jnp.searchsorted(sdst, edges, side="left").astype(jnp.int32)
    bounds = bounds.at[-1].set(e)
    starts = jnp.concatenate([jnp.zeros((1,), jnp.int32), bounds[:-1]])
    cnt = bounds - starts
    nb = _cdiv_arr(cnt)
    pstart = jnp.concatenate([jnp.zeros((1,), jnp.int32),
                              jnp.cumsum(nb).astype(jnp.int32)])
    p = jnp.arange(ep_cap, dtype=jnp.int32)
    b = p // KB
    cid = jnp.clip(jnp.searchsorted(pstart, b, side="right").astype(jnp.int32)
                   - 1, 0, nchunk - 1)
    local = p - pstart[cid] * KB
    eidx = starts[cid] + local
    valid = (local >= 0) & (local < cnt[cid])
    eidx_c = jnp.clip(eidx, 0, e - 1)
    psrc = jnp.where(valid, ssrc[eidx_c], (p * 997) % n_nodes)
    pldst = jnp.where(valid, sdst[eidx_c] - cid * chunk,
                      chunk + (p % NSUB))
    ctrl = (jnp.zeros((nchunk, 16), jnp.int32)
            .at[:, 0].set(pstart[:-1]).at[:, 1].set(nb).reshape(-1))
    return psrc, pldst, ctrl


def _cdiv_arr(x):
    return (x + (KB - 1)) // KB


def _sc_aggregate(x_rows, psrc, pldst, ctrl, chunk, nchunk):
    n, f = x_rows.shape
    g_cnt = f // 128
    npad = nchunk * chunk
    acc_rows = chunk + 128
    nzr = acc_rows // NSUB
    nwr = chunk // NSUB
    ctrl_len = ctrl.shape[0]
    mesh = plsc.VectorSubcoreMesh(core_axis_name="c", subcore_axis_name="s")

    @functools.partial(
        pl.kernel, mesh=mesh,
        out_type=jax.ShapeDtypeStruct((npad, f), jnp.float32),
        scratch_types=[
            pltpu.VMEM((KB,), jnp.int32),
            pltpu.VMEM((KB,), jnp.int32),
            pltpu.VMEM((KB, 128), jnp.float32),
            pltpu.VMEM((NSUB, 128), jnp.float32),
            pltpu.VMEM((ctrl_len,), jnp.int32),
            pltpu.VMEM_SHARED((acc_rows, 128), jnp.float32),
            pltpu.SemaphoreType.DMA,
        ],
    )
    def agg_kernel(x_hbm, psrc_hbm, pldst_hbm, ctrl_hbm, out_hbm,
                   sidx_v, didx_v, rows_v, zero_v, ctrl_v, acc_sh, sem):
        c = lax.axis_index("c")
        s = lax.axis_index("s")
        pltpu.sync_copy(ctrl_hbm, ctrl_v)

        def zinit(i, _):
            zero_v[i // 8, pl.ds((i % 8) * 16, 16)] = jnp.zeros((16,),
                                                                jnp.float32)
            return 0
        lax.fori_loop(0, NSUB * 8, zinit, 0)

        for g in range(g_cnt):
            def chunk_body(ci, _):
                chunk_id = ci * NSC + c
                rec = ctrl_v[pl.ds(chunk_id * 16, 16)]
                base_batch = rec[0]
                nb = rec[1]
                z0 = s * nzr
                for k in range(nzr // NSUB):
                    pltpu.sync_copy(zero_v,
                                    acc_sh.at[pl.ds(z0 + k * NSUB, NSUB)])
                rem = nzr % NSUB
                if rem:
                    pltpu.sync_copy(
                        zero_v.at[pl.ds(0, rem)],
                        acc_sh.at[pl.ds(z0 + (nzr // NSUB) * NSUB, rem)])
                plsc.subcore_barrier()
                nb_s = jnp.maximum(0, (nb - s + NSUB - 1) // NSUB)

                def batch_body(j, _):
                    e0 = (base_batch + s + j * NSUB) * KB
                    pltpu.sync_copy(psrc_hbm.at[pl.ds(e0, KB)], sidx_v)
                    pltpu.sync_copy(pldst_hbm.at[pl.ds(e0, KB)], didx_v)
                    if g_cnt == 1:
                        pltpu.async_copy(x_hbm.at[sidx_v], rows_v, sem).wait()
                    else:
                        pltpu.async_copy(
                            x_hbm.at[sidx_v, pl.ds(g * 128, 128)],
                            rows_v, sem).wait()
                    pltpu.sync_copy(rows_v, acc_sh.at[didx_v], add=True)
                    return 0
                lax.fori_loop(0, nb_s, batch_body, 0)
                plsc.subcore_barrier()
                w0 = s * nwr
                if g_cnt == 1:
                    pltpu.sync_copy(
                        acc_sh.at[pl.ds(w0, nwr)],
                        out_hbm.at[pl.ds(chunk_id * chunk + w0, nwr)])
                else:
                    pltpu.sync_copy(
                        acc_sh.at[pl.ds(w0, nwr)],
                        out_hbm.at[pl.ds(chunk_id * chunk + w0, nwr),
                                   pl.ds(g * 128, 128)])
                plsc.subcore_barrier()
                return 0
            lax.fori_loop(0, nchunk // NSC, chunk_body, 0)

    return agg_kernel(x_rows, psrc, pldst, ctrl)


def _sc_gather(table, idx):
    n, f = table.shape
    b = idx.shape[0]
    per_w = b // (NSC * NSUB)
    mesh = plsc.VectorSubcoreMesh(core_axis_name="c", subcore_axis_name="s")

    @functools.partial(
        pl.kernel, mesh=mesh,
        out_type=jax.ShapeDtypeStruct((b, f), jnp.float32),
        scratch_types=[
            pltpu.VMEM((per_w,), jnp.int32),
            pltpu.VMEM((per_w, f), jnp.float32),
            pltpu.SemaphoreType.DMA,
        ],
    )
    def gather_kernel(tab_hbm, idx_hbm, out_hbm, idx_v, rows_v, sem):
        wid = lax.axis_index("s") * NSC + lax.axis_index("c")
        base = wid * per_w
        pltpu.sync_copy(idx_hbm.at[pl.ds(base, per_w)], idx_v)
        pltpu.async_copy(tab_hbm.at[idx_v], rows_v, sem).wait()
        pltpu.sync_copy(rows_v, out_hbm.at[pl.ds(base, per_w)])

    return gather_kernel(table, idx)


def _tc_stats(x, agg, wbd, bbd, bn, t):
    n, fin = x.shape
    fout = wbd.shape[1]
    grid = n // bn

    def body(xr, ar, wr, br, sums, sqs):
        i = pl.program_id(0)
        h = jnp.maximum(jnp.dot(xr[...] + ar[...], wr[...],
                                preferred_element_type=jnp.float32, precision=jax.lax.Precision.HIGHEST)
                        + br[...], 0.0)

        @pl.when(i == 0)
        def _():
            sums[...] = jnp.zeros((8, fout), jnp.float32)
            sqs[...] = jnp.zeros((8, fout), jnp.float32)
        r = i % 8
        part = jnp.sum(h, axis=0, keepdims=True)
        part2 = jnp.sum(h * h, axis=0, keepdims=True)
        sums[pl.ds(r, 1), :] += part
        sqs[pl.ds(r, 1), :] += part2

    return pl.pallas_call(
        body,
        grid=(grid,),
        in_specs=[
            pl.BlockSpec((bn, fin), lambda i: (i, 0)),
            pl.BlockSpec((bn, fin), lambda i: (i, 0)),
            pl.BlockSpec((fin, fout), lambda i: (0, 0)),
            pl.BlockSpec((1, fout), lambda i: (0, 0)),
        ],
        out_specs=[
            pl.BlockSpec((8, fout), lambda i: (0, 0)),
            pl.BlockSpec((8, fout), lambda i: (0, 0)),
        ],
        out_shape=[
            jax.ShapeDtypeStruct((8, fout), jnp.float32),
            jax.ShapeDtypeStruct((8, fout), jnp.float32),
        ],
    )(x, agg, wbd, bbd)


def _tc_main(x, agg, sums, sqs, wbd, bbd, g_t, be_t, wb, bb,
             wih_t, whh_t, bih, bhh, bn, t, h, emit_all):
    n, fin = x.shape
    fout = wbd.shape[1]
    n_total = float(n)
    grid = n // bn
    out_w = fout if emit_all else h

    def body(xr, ar, sums_r, sqs_r, wr, br, gr, ber, wbr, bbr,
             wihr, whhr, bihr, bhhr, out):
        hpre = jnp.maximum(jnp.dot(xr[...] + ar[...], wr[...],
                                   preferred_element_type=jnp.float32, precision=jax.lax.Precision.HIGHEST)
                           + br[...], 0.0)
        tot = jnp.sum(sums_r[...], axis=0, keepdims=True)
        tot2 = jnp.sum(sqs_r[...], axis=0, keepdims=True)
        mean = tot / n_total
        var = tot2 / n_total - mean * mean
        rstd = lax.rsqrt(var + 1e-5)
        scale = rstd * gr[...]
        shift = ber[...] - mean * scale
        hn = hpre * scale + shift
        carry = jnp.zeros((bn, h), jnp.float32)
        for ti in range(t):
            ht = hn[:, ti * h:(ti + 1) * h]
            h1 = jnp.maximum(jnp.dot(ht, wbr[...],
                                     preferred_element_type=jnp.float32, precision=jax.lax.Precision.HIGHEST)
                             + bbr[...], 0.0)
            gi = jnp.dot(h1, wihr[...],
                         preferred_element_type=jnp.float32, precision=jax.lax.Precision.HIGHEST) + bihr[...]
            gh = jnp.dot(carry, whhr[...],
                         preferred_element_type=jnp.float32, precision=jax.lax.Precision.HIGHEST) + bhhr[...]
            r = jax.nn.sigmoid(gi[:, 0:h] + gh[:, 0:h])
            z = jax.nn.sigmoid(gi[:, h:2 * h] + gh[:, h:2 * h])
            nn = jnp.tanh(gi[:, 2 * h:3 * h] + r * gh[:, 2 * h:3 * h])
            carry = (1.0 - z) * nn + z * carry
            if emit_all:
                out[:, ti * h:(ti + 1) * h] = carry
        if not emit_all:
            out[...] = carry

    return pl.pallas_call(
        body,
        grid=(grid,),
        in_specs=[
            pl.BlockSpec((bn, fin), lambda i: (i, 0)),
            pl.BlockSpec((bn, fin), lambda i: (i, 0)),
            pl.BlockSpec((8, fout), lambda i: (0, 0)),
            pl.BlockSpec((8, fout), lambda i: (0, 0)),
            pl.BlockSpec((fin, fout), lambda i: (0, 0)),
            pl.BlockSpec((1, fout), lambda i: (0, 0)),
            pl.BlockSpec((1, fout), lambda i: (0, 0)),
            pl.BlockSpec((1, fout), lambda i: (0, 0)),
            pl.BlockSpec((h, h), lambda i: (0, 0)),
            pl.BlockSpec((1, h), lambda i: (0, 0)),
            pl.BlockSpec((h, 3 * h), lambda i: (0, 0)),
            pl.BlockSpec((h, 3 * h), lambda i: (0, 0)),
            pl.BlockSpec((1, 3 * h), lambda i: (0, 0)),
            pl.BlockSpec((1, 3 * h), lambda i: (0, 0)),
        ],
        out_specs=pl.BlockSpec((bn, out_w), lambda i: (i, 0)),
        out_shape=jax.ShapeDtypeStruct((n, out_w), jnp.float32),
    )(x, agg, sums, sqs, wbd, bbd, g_t, be_t, wb, bb,
      wih_t, whh_t, bih, bhh)


def _tc_head(hg, apt, wf1a, wf1b, bfc1, wfc2, bfc2, wfc3, bfc3):
    b, h = hg.shape
    h2 = wfc2.shape[1]

    def body(hgr, aptr, w1ar, w1br, b1r, w2r, b2r, w3r, b3r, out):
        y = (jnp.dot(hgr[...], w1ar[...], preferred_element_type=jnp.float32, precision=jax.lax.Precision.HIGHEST)
             + jnp.dot(aptr[...], w1br[...], preferred_element_type=jnp.float32, precision=jax.lax.Precision.HIGHEST)
             + b1r[...])
        y = jnp.where(y > 0, y, 0.1 * y)
        y = jnp.dot(y, w2r[...], preferred_element_type=jnp.float32, precision=jax.lax.Precision.HIGHEST) + b2r[...]
        y = jnp.where(y > 0, y, 0.05 * y)
        out[...] = (jnp.dot(y, w3r[...], preferred_element_type=jnp.float32, precision=jax.lax.Precision.HIGHEST)
                    + b3r[...])

    apt_w = apt.shape[1]
    h3 = wfc3.shape[0]
    return pl.pallas_call(
        body,
        out_shape=jax.ShapeDtypeStruct((b, 128), jnp.float32),
    )(hg, apt, wf1a, wf1b, bfc1.reshape(1, -1), wfc2,
      bfc2.reshape(1, -1), jnp.pad(wfc3, ((0, 0), (0, 127))),
      jnp.pad(bfc3, (0, 127)).reshape(1, -1))


def _block_diag(w, t):
    a, b = w.shape
    eye = jnp.eye(t, dtype=w.dtype)
    return (eye[:, None, :, None] * w[None, :, None, :]).reshape(t * a, t * b)


def kernel(x_seq, edge_index, node_idx, apart_feature, W1a, b1a, g1a, be1a,
           W1b, b1b, W2a, b2a, g2a, be2a, W2b, b2b, Wih, Whh, bih, bhh,
           Wfc1, bfc1, Wfc2, bfc2, Wfc3, bfc3):
    t, n, in_dim = x_seq.shape
    h = W1a.shape[1]
    chunk = 4096
    nchunk = _cdiv(n, chunk)
    if nchunk % NSC:
        nchunk += 1
    bn = 1000 if n % 1000 == 0 else n

    src, dst = edge_index[0], edge_index[1]
    psrc, pldst, ctrl = _prep_edges(src, dst, n, chunk, nchunk)

    f1 = t * in_dim
    f1p = _cdiv(f1, 128) * 128
    xp = jnp.pad(jnp.transpose(x_seq, (1, 0, 2)).reshape(n, f1),
                 ((0, 0), (0, f1p - f1)))

    w1a_bd = jnp.pad(_block_diag(W1a, t), ((0, f1p - f1), (0, 0)))
    b1a_t = jnp.tile(b1a, t).reshape(1, -1)
    g1a_t = jnp.tile(g1a, t).reshape(1, -1)
    be1a_t = jnp.tile(be1a, t).reshape(1, -1)
    w2a_bd = _block_diag(W2a, t)
    b2a_t = jnp.tile(b2a, t).reshape(1, -1)
    g2a_t = jnp.tile(g2a, t).reshape(1, -1)
    be2a_t = jnp.tile(be2a, t).reshape(1, -1)
    wih_t = Wih.T
    whh_t = Whh.T
    bih_r = bih.reshape(1, -1)
    bhh_r = bhh.reshape(1, -1)
    b1b_r = b1b.reshape(1, -1)
    b2b_r = b2b.reshape(1, -1)

    agg1 = _sc_aggregate(xp, psrc, pldst, ctrl, chunk, nchunk)[:n]
    s1, q1 = _tc_stats(xp, agg1, w1a_bd, b1a_t, bn, t)
    outs = _tc_main(xp, agg1, s1, q1, w1a_bd, b1a_t, g1a_t, be1a_t,
                    W1b, b1b_r, wih_t, whh_t, bih_r, bhh_r,
                    bn, t, h, emit_all=True)

    agg2 = _sc_aggregate(outs, psrc, pldst, ctrl, chunk, nchunk)[:n]
    s2, q2 = _tc_stats(outs, agg2, w2a_bd, b2a_t, bn, t)
    h_last = _tc_main(outs, agg2, s2, q2, w2a_bd, b2a_t, g2a_t, be2a_t,
                      W2b, b2b_r, wih_t, whh_t, bih_r, bhh_r,
                      bn, t, h, emit_all=False)

    hg = _sc_gather(jnp.pad(h_last, ((0, 0), (0, 128 - h))), node_idx)[:, :h]
    y = _tc_head(hg, apart_feature, Wfc1[:h], Wfc1[h:], bfc1,
                 Wfc2, bfc2, Wfc3, bfc3)
    return y[:, :1]


# speedup vs baseline: 26.5672x; 1.0223x over previous
import functools
import math

import jax
import jax.numpy as jnp
from jax import lax
from jax.experimental import pallas as pl
from jax.experimental.pallas import tpu as pltpu
from jax.experimental.pallas import tpu_sc as plsc

KB = 128
NSC = 2
NSUB = 16
NW = NSC * NSUB


def _cdiv(a, b):
    return (a + b - 1) // b


def _prep_edges(src, dst, n_nodes, chunk, nchunk):
    e = src.shape[0]
    ep_cap = _cdiv(e + nchunk * KB * 8 + (256 + SB) * KB, KB) * KB
    perm = jnp.argsort(dst)
    sdst = dst[perm]
    ssrc = src[perm]
    chunk_of = jnp.minimum(sdst // chunk, nchunk - 1)
    edges = jnp.arange(1, nchunk + 1, dtype=jnp.int32) * chunk
    bounds = jnp.searchsorted(sdst, edges, side="left").astype(jnp.int32)
    bounds = bounds.at[-1].set(e)
    starts = jnp.concatenate([jnp.zeros((1,), jnp.int32), bounds[:-1]])
    cnt = bounds - starts
    nb = ((_cdiv_arr(cnt) + 7) // 8) * 8
    pstart = jnp.concatenate([jnp.zeros((1,), jnp.int32),
                              jnp.cumsum(nb).astype(jnp.int32)])
    p = jnp.arange(ep_cap, dtype=jnp.int32)
    b = p // KB
    cid = jnp.clip(jnp.searchsorted(pstart, b, side="right").astype(jnp.int32)
                   - 1, 0, nchunk - 1)
    local = p - pstart[cid] * KB
    eidx = starts[cid] + local
    valid = (local >= 0) & (local < cnt[cid])
    eidx_c = jnp.clip(eidx, 0, e - 1)
    psrc = jnp.where(valid, ssrc[eidx_c], (p * 997) % n_nodes)
    pldst = jnp.where(valid, sdst[eidx_c] - cid * chunk,
                      chunk + (p % NSUB))
    ctrl = (jnp.zeros((nchunk, 16), jnp.int32)
            .at[:, 0].set(pstart[:-1]).at[:, 1].set(nb).reshape(-1))
    return psrc, pldst, ctrl


def _cdiv_arr(x):
    return (x + (KB - 1)) // KB


SB = 16


def _sc_aggregate(x_rows, psrc, pldst, ctrl, chunk, nchunk):
    n, f = x_rows.shape
    g_cnt = f // 128
    npad = nchunk * chunk
    acc_rows = chunk + 128
    nzr = acc_rows // NSUB
    nwr = chunk // NSUB
    ctrl_len = ctrl.shape[0]
    nbat_cap = psrc.shape[0] // KB
    psrc2 = psrc.reshape(nbat_cap, KB)
    pldst2 = pldst.reshape(nbat_cap, KB)
    mesh = plsc.VectorSubcoreMesh(core_axis_name="c", subcore_axis_name="s")

    @functools.partial(
        pl.kernel, mesh=mesh,
        out_type=jax.ShapeDtypeStruct((npad, f), jnp.float32),
        scratch_types=(
            [pltpu.VMEM((SB, KB), jnp.int32),
             pltpu.VMEM((SB, KB), jnp.int32),
             pltpu.VMEM((NSUB, 128), jnp.float32),
             pltpu.VMEM((ctrl_len,), jnp.int32)]
            + [pltpu.VMEM((KB, 128), jnp.float32) for _ in range(g_cnt)]
            + [pltpu.VMEM_SHARED((acc_rows, 128), jnp.float32)
               for _ in range(g_cnt)]
            + [pltpu.SemaphoreType.DMA]
        ),
    )
    def agg_kernel(x_hbm, psrc_hbm, pldst_hbm, ctrl_hbm, out_hbm,
                   sidx_st, didx_st, zero_v, ctrl_v, *rest):
        rows_b = rest[:g_cnt]
        accs = rest[g_cnt:2 * g_cnt]
        sem = rest[2 * g_cnt]
        c = lax.axis_index("c")
        s = lax.axis_index("s")
        pltpu.sync_copy(ctrl_hbm, ctrl_v)

        def zinit(i, _):
            zero_v[i // 8, pl.ds((i % 8) * 16, 16)] = jnp.zeros((16,),
                                                                jnp.float32)
            return 0
        lax.fori_loop(0, NSUB * 8, zinit, 0)

        def chunk_body(ci, _):
            chunk_id = ci * NSC + c
            rec = ctrl_v[pl.ds(chunk_id * 16, 16)]
            base_batch = rec[0]
            nb = rec[1]
            z0 = s * nzr
            for acc in accs:
                for k in range(nzr // NSUB):
                    pltpu.sync_copy(zero_v,
                                    acc.at[pl.ds(z0 + k * NSUB, NSUB)])
                rem = nzr % NSUB
                if rem:
                    pltpu.sync_copy(
                        zero_v.at[pl.ds(0, rem)],
                        acc.at[pl.ds(z0 + (nzr // NSUB) * NSUB, rem)])
            plsc.subcore_barrier()
            qs = (((nb + NSUB - 1) // NSUB + 7) // 8) * 8
            b0 = base_batch + s * qs
            myb = jnp.clip(nb - s * qs, 0, qs)

            def super_body(sb, _):
                off = pl.multiple_of(b0 + sb * SB, 8)
                pltpu.sync_copy(psrc_hbm.at[pl.ds(off, SB)], sidx_st)
                pltpu.sync_copy(pldst_hbm.at[pl.ds(off, SB)], didx_st)
                jb = jnp.minimum(SB, myb - sb * SB)

                def batch_body(j, _):
                    cps = []
                    for g in range(g_cnt):
                        if g_cnt == 1:
                            cps.append(pltpu.async_copy(
                                x_hbm.at[sidx_st.at[j]], rows_b[g], sem))
                        else:
                            cps.append(pltpu.async_copy(
                                x_hbm.at[sidx_st.at[j],
                                         pl.ds(g * 128, 128)],
                                rows_b[g], sem))
                    for g in range(g_cnt):
                        cps[g].wait()
                        pltpu.sync_copy(rows_b[g], accs[g].at[didx_st.at[j]],
                                        add=True)
                    return 0
                lax.fori_loop(0, jb, batch_body, 0)
                return 0
            lax.fori_loop(0, (myb + SB - 1) // SB, super_body, 0)
            plsc.subcore_barrier()
            w0 = s * nwr
            for g in range(g_cnt):
                if g_cnt == 1:
                    pltpu.sync_copy(
                        accs[g].at[pl.ds(w0, nwr)],
                        out_hbm.at[pl.ds(chunk_id * chunk + w0, nwr)])
                else:
                    pltpu.sync_copy(
                        accs[g].at[pl.ds(w0, nwr)],
                        out_hbm.at[pl.ds(chunk_id * chunk + w0, nwr),
                                   pl.ds(g * 128, 128)])
            plsc.subcore_barrier()
            return 0
        lax.fori_loop(0, nchunk // NSC, chunk_body, 0)

    return agg_kernel(x_rows, psrc2, pldst2, ctrl)


def _sc_gather(table, idx):
    n, f = table.shape
    b = idx.shape[0]
    per_w = b // (NSC * NSUB)
    mesh = plsc.VectorSubcoreMesh(core_axis_name="c", subcore_axis_name="s")

    @functools.partial(
        pl.kernel, mesh=mesh,
        out_type=jax.ShapeDtypeStruct((b, f), jnp.float32),
        scratch_types=[
            pltpu.VMEM((per_w,), jnp.int32),
            pltpu.VMEM((per_w, f), jnp.float32),
            pltpu.SemaphoreType.DMA,
        ],
    )
    def gather_kernel(tab_hbm, idx_hbm, out_hbm, idx_v, rows_v, sem):
        wid = lax.axis_index("s") * NSC + lax.axis_index("c")
        base = wid * per_w
        pltpu.sync_copy(idx_hbm.at[pl.ds(base, per_w)], idx_v)
        pltpu.async_copy(tab_hbm.at[idx_v], rows_v, sem).wait()
        pltpu.sync_copy(rows_v, out_hbm.at[pl.ds(base, per_w)])

    return gather_kernel(table, idx)


def _tc_stats(x, agg, wbd, bbd, bn, t):
    n, fin = x.shape
    fout = wbd.shape[1]
    grid = n // bn

    def body(xr, ar, wr, br, sums, sqs):
        i = pl.program_id(0)
        h = jnp.maximum(jnp.dot(xr[...] + ar[...], wr[...],
                                preferred_element_type=jnp.float32, precision=jax.lax.Precision.HIGHEST)
                        + br[...], 0.0)

        @pl.when(i == 0)
        def _():
            sums[...] = jnp.zeros((8, fout), jnp.float32)
            sqs[...] = jnp.zeros((8, fout), jnp.float32)
        r = i % 8
        part = jnp.sum(h, axis=0, keepdims=True)
        part2 = jnp.sum(h * h, axis=0, keepdims=True)
        sums[pl.ds(r, 1), :] += part
        sqs[pl.ds(r, 1), :] += part2

    return pl.pallas_call(
        body,
        grid=(grid,),
        in_specs=[
            pl.BlockSpec((bn, fin), lambda i: (i, 0)),
            pl.BlockSpec((bn, fin), lambda i: (i, 0)),
            pl.BlockSpec((fin, fout), lambda i: (0, 0)),
            pl.BlockSpec((1, fout), lambda i: (0, 0)),
        ],
        out_specs=[
            pl.BlockSpec((8, fout), lambda i: (0, 0)),
            pl.BlockSpec((8, fout), lambda i: (0, 0)),
        ],
        out_shape=[
            jax.ShapeDtypeStruct((8, fout), jnp.float32),
            jax.ShapeDtypeStruct((8, fout), jnp.float32),
        ],
    )(x, agg, wbd, bbd)


def _tc_main(x, agg, sums, sqs, wbd, bbd, g_t, be_t, wb, bb,
             wih_t, whh_t, bih, bhh, bn, t, h, emit_all):
    n, fin = x.shape
    fout = wbd.shape[1]
    n_total = float(n)
    grid = n // bn
    out_w = fout if emit_all else h

    def body(xr, ar, sums_r, sqs_r, wr, br, gr, ber, wbr, bbr,
             wihr, whhr, bihr, bhhr, out):
        hpre = jnp.maximum(jnp.dot(xr[...] + ar[...], wr[...],
                                   preferred_element_type=jnp.float32, precision=jax.lax.Precision.HIGHEST)
                           + br[...], 0.0)
        tot = jnp.sum(sums_r[...], axis=0, keepdims=True)
        tot2 = jnp.sum(sqs_r[...], axis=0, keepdims=True)
        mean = tot / n_total
        var = tot2 / n_total - mean * mean
        rstd = lax.rsqrt(var + 1e-5)
        scale = rstd * gr[...]
        shift = ber[...] - mean * scale
        hn = hpre * scale + shift
        carry = jnp.zeros((bn, h), jnp.float32)
        for ti in range(t):
            ht = hn[:, ti * h:(ti + 1) * h]
            h1 = jnp.maximum(jnp.dot(ht, wbr[...],
                                     preferred_element_type=jnp.float32, precision=jax.lax.Precision.HIGHEST)
                             + bbr[...], 0.0)
            gi = jnp.dot(h1, wihr[...],
                         preferred_element_type=jnp.float32, precision=jax.lax.Precision.HIGHEST) + bihr[...]
            gh = jnp.dot(carry, whhr[...],
                         preferred_element_type=jnp.float32, precision=jax.lax.Precision.HIGHEST) + bhhr[...]
            r = jax.nn.sigmoid(gi[:, 0:h] + gh[:, 0:h])
            z = jax.nn.sigmoid(gi[:, h:2 * h] + gh[:, h:2 * h])
            nn = jnp.tanh(gi[:, 2 * h:3 * h] + r * gh[:, 2 * h:3 * h])
            carry = (1.0 - z) * nn + z * carry
            if emit_all:
                out[:, ti * h:(ti + 1) * h] = carry
        if not emit_all:
            out[...] = carry

    return pl.pallas_call(
        body,
        grid=(grid,),
        in_specs=[
            pl.BlockSpec((bn, fin), lambda i: (i, 0)),
            pl.BlockSpec((bn, fin), lambda i: (i, 0)),
            pl.BlockSpec((8, fout), lambda i: (0, 0)),
            pl.BlockSpec((8, fout), lambda i: (0, 0)),
            pl.BlockSpec((fin, fout), lambda i: (0, 0)),
            pl.BlockSpec((1, fout), lambda i: (0, 0)),
            pl.BlockSpec((1, fout), lambda i: (0, 0)),
            pl.BlockSpec((1, fout), lambda i: (0, 0)),
            pl.BlockSpec((h, h), lambda i: (0, 0)),
            pl.BlockSpec((1, h), lambda i: (0, 0)),
            pl.BlockSpec((h, 3 * h), lambda i: (0, 0)),
            pl.BlockSpec((h, 3 * h), lambda i: (0, 0)),
            pl.BlockSpec((1, 3 * h), lambda i: (0, 0)),
            pl.BlockSpec((1, 3 * h), lambda i: (0, 0)),
        ],
        out_specs=pl.BlockSpec((bn, out_w), lambda i: (i, 0)),
        out_shape=jax.ShapeDtypeStruct((n, out_w), jnp.float32),
    )(x, agg, sums, sqs, wbd, bbd, g_t, be_t, wb, bb,
      wih_t, whh_t, bih, bhh)


def _tc_head(hg, apt, wf1a, wf1b, bfc1, wfc2, bfc2, wfc3, bfc3):
    b, h = hg.shape
    h2 = wfc2.shape[1]

    def body(hgr, aptr, w1ar, w1br, b1r, w2r, b2r, w3r, b3r, out):
        y = (jnp.dot(hgr[...], w1ar[...], preferred_element_type=jnp.float32, precision=jax.lax.Precision.HIGHEST)
             + jnp.dot(aptr[...], w1br[...], preferred_element_type=jnp.float32, precision=jax.lax.Precision.HIGHEST)
             + b1r[...])
        y = jnp.where(y > 0, y, 0.1 * y)
        y = jnp.dot(y, w2r[...], preferred_element_type=jnp.float32, precision=jax.lax.Precision.HIGHEST) + b2r[...]
        y = jnp.where(y > 0, y, 0.05 * y)
        out[...] = (jnp.dot(y, w3r[...], preferred_element_type=jnp.float32, precision=jax.lax.Precision.HIGHEST)
                    + b3r[...])

    apt_w = apt.shape[1]
    h3 = wfc3.shape[0]
    return pl.pallas_call(
        body,
        out_shape=jax.ShapeDtypeStruct((b, 128), jnp.float32),
    )(hg, apt, wf1a, wf1b, bfc1.reshape(1, -1), wfc2,
      bfc2.reshape(1, -1), jnp.pad(wfc3, ((0, 0), (0, 127))),
      jnp.pad(bfc3, (0, 127)).reshape(1, -1))


def _block_diag(w, t):
    a, b = w.shape
    eye = jnp.eye(t, dtype=w.dtype)
    return (eye[:, None, :, None] * w[None, :, None, :]).reshape(t * a, t * b)


def kernel(x_seq, edge_index, node_idx, apart_feature, W1a, b1a, g1a, be1a,
           W1b, b1b, W2a, b2a, g2a, be2a, W2b, b2b, Wih, Whh, bih, bhh,
           Wfc1, bfc1, Wfc2, bfc2, Wfc3, bfc3):
    t, n, in_dim = x_seq.shape
    h = W1a.shape[1]
    chunk = 1536
    nchunk = _cdiv(n, chunk)
    if nchunk % NSC:
        nchunk += 1
    bn = 1000 if n % 1000 == 0 else n

    src, dst = edge_index[0], edge_index[1]
    psrc, pldst, ctrl = _prep_edges(src, dst, n, chunk, nchunk)

    f1 = t * in_dim
    f1p = _cdiv(f1, 128) * 128
    xp = jnp.pad(jnp.transpose(x_seq, (1, 0, 2)).reshape(n, f1),
                 ((0, 0), (0, f1p - f1)))

    w1a_bd = jnp.pad(_block_diag(W1a, t), ((0, f1p - f1), (0, 0)))
    b1a_t = jnp.tile(b1a, t).reshape(1, -1)
    g1a_t = jnp.tile(g1a, t).reshape(1, -1)
    be1a_t = jnp.tile(be1a, t).reshape(1, -1)
    w2a_bd = _block_diag(W2a, t)
    b2a_t = jnp.tile(b2a, t).reshape(1, -1)
    g2a_t = jnp.tile(g2a, t).reshape(1, -1)
    be2a_t = jnp.tile(be2a, t).reshape(1, -1)
    wih_t = Wih.T
    whh_t = Whh.T
    bih_r = bih.reshape(1, -1)
    bhh_r = bhh.reshape(1, -1)
    b1b_r = b1b.reshape(1, -1)
    b2b_r = b2b.reshape(1, -1)

    agg1 = _sc_aggregate(xp, psrc, pldst, ctrl, chunk, nchunk)[:n]
    s1, q1 = _tc_stats(xp, agg1, w1a_bd, b1a_t, bn, t)
    outs = _tc_main(xp, agg1, s1, q1, w1a_bd, b1a_t, g1a_t, be1a_t,
                    W1b, b1b_r, wih_t, whh_t, bih_r, bhh_r,
                    bn, t, h, emit_all=True)

    agg2 = _sc_aggregate(outs, psrc, pldst, ctrl, chunk, nchunk)[:n]
    s2, q2 = _tc_stats(outs, agg2, w2a_bd, b2a_t, bn, t)
    h_last = _tc_main(outs, agg2, s2, q2, w2a_bd, b2a_t, g2a_t, be2a_t,
                      W2b, b2b_r, wih_t, whh_t, bih_r, bhh_r,
                      bn, t, h, emit_all=False)

    hg = _sc_gather(jnp.pad(h_last, ((0, 0), (0, 128 - h))), node_idx)[:, :h]
    y = _tc_head(hg, apart_feature, Wfc1[:h], Wfc1[h:], bfc1,
                 Wfc2, bfc2, Wfc3, bfc3)
    return y[:, :1]


# speedup vs baseline: 34.6435x; 1.3040x over previous
import functools
import math

import jax
import jax.numpy as jnp
from jax import lax
from jax.experimental import pallas as pl
from jax.experimental.pallas import tpu as pltpu
from jax.experimental.pallas import tpu_sc as plsc

KB = 128
NSC = 2
NSUB = 16
NW = NSC * NSUB


def _cdiv(a, b):
    return (a + b - 1) // b


def _prep_edges(src, dst, n_nodes, chunk, nchunk):
    e = src.shape[0]
    ep_cap = _cdiv(e + nchunk * KB * 8 + (256 + SB) * KB, KB) * KB
    perm = jnp.argsort(dst)
    sdst = dst[perm]
    ssrc = src[perm]
    chunk_of = jnp.minimum(sdst // chunk, nchunk - 1)
    edges = jnp.arange(1, nchunk + 1, dtype=jnp.int32) * chunk
    bounds = jnp.searchsorted(sdst, edges, side="left").astype(jnp.int32)
    bounds = bounds.at[-1].set(e)
    starts = jnp.concatenate([jnp.zeros((1,), jnp.int32), bounds[:-1]])
    cnt = bounds - starts
    nb = ((_cdiv_arr(cnt) + 7) // 8) * 8
    pstart = jnp.concatenate([jnp.zeros((1,), jnp.int32),
                              jnp.cumsum(nb).astype(jnp.int32)])
    p = jnp.arange(ep_cap, dtype=jnp.int32)
    nbat = ep_cap // KB
    cid_b = jnp.clip(
        jnp.searchsorted(pstart, jnp.arange(nbat, dtype=jnp.int32),
                         side="right").astype(jnp.int32) - 1, 0, nchunk - 1)
    cid = jnp.repeat(cid_b, KB)
    local = p - pstart[cid] * KB
    eidx = starts[cid] + local
    valid = (local >= 0) & (local < cnt[cid])
    eidx_c = jnp.clip(eidx, 0, e - 1)
    psrc = jnp.where(valid, ssrc[eidx_c], (p * 997) % n_nodes)
    pldst = jnp.where(valid, sdst[eidx_c] - cid * chunk,
                      chunk + (p % NSUB))
    ctrl = (jnp.zeros((nchunk, 16), jnp.int32)
            .at[:, 0].set(pstart[:-1]).at[:, 1].set(nb).reshape(-1))
    return psrc, pldst, ctrl


def _cdiv_arr(x):
    return (x + (KB - 1)) // KB


SB = 16


def _sc_aggregate(x_rows, psrc, pldst, ctrl, chunk, nchunk):
    n, f = x_rows.shape
    g_cnt = f // 128
    npad = nchunk * chunk
    acc_rows = chunk + 128
    nzr = acc_rows // NSUB
    nwr = chunk // NSUB
    ctrl_len = ctrl.shape[0]
    nbat_cap = psrc.shape[0] // KB
    psrc2 = psrc.reshape(nbat_cap, KB)
    pldst2 = pldst.reshape(nbat_cap, KB)
    mesh = plsc.VectorSubcoreMesh(core_axis_name="c", subcore_axis_name="s")

    @functools.partial(
        pl.kernel, mesh=mesh,
        out_type=jax.ShapeDtypeStruct((npad, f), jnp.float32),
        scratch_types=(
            [pltpu.VMEM((SB, KB), jnp.int32),
             pltpu.VMEM((SB, KB), jnp.int32),
             pltpu.VMEM((NSUB, 128), jnp.float32),
             pltpu.VMEM((ctrl_len,), jnp.int32)]
            + [pltpu.VMEM((KB, 128), jnp.float32) for _ in range(g_cnt)]
            + [pltpu.VMEM_SHARED((acc_rows, 128), jnp.float32)
               for _ in range(g_cnt)]
            + [pltpu.SemaphoreType.DMA]
        ),
    )
    def agg_kernel(x_hbm, psrc_hbm, pldst_hbm, ctrl_hbm, out_hbm,
                   sidx_st, didx_st, zero_v, ctrl_v, *rest):
        rows_b = rest[:g_cnt]
        accs = rest[g_cnt:2 * g_cnt]
        sem = rest[2 * g_cnt]
        c = lax.axis_index("c")
        s = lax.axis_index("s")
        pltpu.sync_copy(ctrl_hbm, ctrl_v)

        def zinit(i, _):
            zero_v[i // 8, pl.ds((i % 8) * 16, 16)] = jnp.zeros((16,),
                                                                jnp.float32)
            return 0
        lax.fori_loop(0, NSUB * 8, zinit, 0)

        def chunk_body(ci, _):
            chunk_id = ci * NSC + c
            rec = ctrl_v[pl.ds(chunk_id * 16, 16)]
            base_batch = rec[0]
            nb = rec[1]
            z0 = s * nzr
            for acc in accs:
                for k in range(nzr // NSUB):
                    pltpu.sync_copy(zero_v,
                                    acc.at[pl.ds(z0 + k * NSUB, NSUB)])
                rem = nzr % NSUB
                if rem:
                    pltpu.sync_copy(
                        zero_v.at[pl.ds(0, rem)],
                        acc.at[pl.ds(z0 + (nzr // NSUB) * NSUB, rem)])
            plsc.subcore_barrier()
            qs = (((nb + NSUB - 1) // NSUB + 7) // 8) * 8
            b0 = base_batch + s * qs
            myb = jnp.clip(nb - s * qs, 0, qs)

            def super_body(sb, _):
                off = pl.multiple_of(b0 + sb * SB, 8)
                pltpu.sync_copy(psrc_hbm.at[pl.ds(off, SB)], sidx_st)
                pltpu.sync_copy(pldst_hbm.at[pl.ds(off, SB)], didx_st)
                jb = jnp.minimum(SB, myb - sb * SB)

                def batch_body(j, _):
                    cps = []
                    for g in range(g_cnt):
                        if g_cnt == 1:
                            cps.append(pltpu.async_copy(
                                x_hbm.at[sidx_st.at[j]], rows_b[g], sem))
                        else:
                            cps.append(pltpu.async_copy(
                                x_hbm.at[sidx_st.at[j],
                                         pl.ds(g * 128, 128)],
                                rows_b[g], sem))
                    for g in range(g_cnt):
                        cps[g].wait()
                        pltpu.sync_copy(rows_b[g], accs[g].at[didx_st.at[j]],
                                        add=True)
                    return 0
                lax.fori_loop(0, jb, batch_body, 0)
                return 0
            lax.fori_loop(0, (myb + SB - 1) // SB, super_body, 0)
            plsc.subcore_barrier()
            w0 = s * nwr
            for g in range(g_cnt):
                if g_cnt == 1:
                    pltpu.sync_copy(
                        accs[g].at[pl.ds(w0, nwr)],
                        out_hbm.at[pl.ds(chunk_id * chunk + w0, nwr)])
                else:
                    pltpu.sync_copy(
                        accs[g].at[pl.ds(w0, nwr)],
                        out_hbm.at[pl.ds(chunk_id * chunk + w0, nwr),
                                   pl.ds(g * 128, 128)])
            plsc.subcore_barrier()
            return 0
        lax.fori_loop(0, nchunk // NSC, chunk_body, 0)

    return agg_kernel(x_rows, psrc2, pldst2, ctrl)


def _sc_gather(table, idx):
    n, f = table.shape
    b = idx.shape[0]
    per_w = b // (NSC * NSUB)
    mesh = plsc.VectorSubcoreMesh(core_axis_name="c", subcore_axis_name="s")

    @functools.partial(
        pl.kernel, mesh=mesh,
        out_type=jax.ShapeDtypeStruct((b, f), jnp.float32),
        scratch_types=[
            pltpu.VMEM((per_w,), jnp.int32),
            pltpu.VMEM((per_w, f), jnp.float32),
            pltpu.SemaphoreType.DMA,
        ],
    )
    def gather_kernel(tab_hbm, idx_hbm, out_hbm, idx_v, rows_v, sem):
        wid = lax.axis_index("s") * NSC + lax.axis_index("c")
        base = wid * per_w
        pltpu.sync_copy(idx_hbm.at[pl.ds(base, per_w)], idx_v)
        pltpu.async_copy(tab_hbm.at[idx_v], rows_v, sem).wait()
        pltpu.sync_copy(rows_v, out_hbm.at[pl.ds(base, per_w)])

    return gather_kernel(table, idx)


def _tc_stats(x, agg, wbd, bbd, bn, t):
    n, fin = x.shape
    fout = wbd.shape[1]
    grid = n // bn

    def body(xr, ar, wr, br, sums, sqs):
        i = pl.program_id(0)
        h = jnp.maximum(jnp.dot(xr[...] + ar[...], wr[...],
                                preferred_element_type=jnp.float32)
                        + br[...], 0.0)

        @pl.when(i == 0)
        def _():
            sums[...] = jnp.zeros((8, fout), jnp.float32)
            sqs[...] = jnp.zeros((8, fout), jnp.float32)
        r = i % 8
        part = jnp.sum(h, axis=0, keepdims=True)
        part2 = jnp.sum(h * h, axis=0, keepdims=True)
        sums[pl.ds(r, 1), :] += part
        sqs[pl.ds(r, 1), :] += part2

    return pl.pallas_call(
        body,
        grid=(grid,),
        in_specs=[
            pl.BlockSpec((bn, fin), lambda i: (i, 0)),
            pl.BlockSpec((bn, fin), lambda i: (i, 0)),
            pl.BlockSpec((fin, fout), lambda i: (0, 0)),
            pl.BlockSpec((1, fout), lambda i: (0, 0)),
        ],
        out_specs=[
            pl.BlockSpec((8, fout), lambda i: (0, 0)),
            pl.BlockSpec((8, fout), lambda i: (0, 0)),
        ],
        out_shape=[
            jax.ShapeDtypeStruct((8, fout), jnp.float32),
            jax.ShapeDtypeStruct((8, fout), jnp.float32),
        ],
    )(x, agg, wbd, bbd)


def _tc_main(x, agg, sums, sqs, wbd, bbd, g_t, be_t, wb, bb,
             wih_t, whh_t, bih, bhh, bn, t, h, emit_all):
    n, fin = x.shape
    fout = wbd.shape[1]
    n_total = float(n)
    grid = n // bn
    out_w = fout if emit_all else h

    def body(xr, ar, sums_r, sqs_r, wr, br, gr, ber, wbr, bbr,
             wihr, whhr, bihr, bhhr, out):
        hpre = jnp.maximum(jnp.dot(xr[...] + ar[...], wr[...],
                                   preferred_element_type=jnp.float32)
                           + br[...], 0.0)
        tot = jnp.sum(sums_r[...], axis=0, keepdims=True)
        tot2 = jnp.sum(sqs_r[...], axis=0, keepdims=True)
        mean = tot / n_total
        var = tot2 / n_total - mean * mean
        rstd = lax.rsqrt(var + 1e-5)
        scale = rstd * gr[...]
        shift = ber[...] - mean * scale
        hn = hpre * scale + shift
        carry = jnp.zeros((bn, h), jnp.float32)
        for ti in range(t):
            ht = hn[:, ti * h:(ti + 1) * h]
            h1 = jnp.maximum(jnp.dot(ht, wbr[...],
                                     preferred_element_type=jnp.float32)
                             + bbr[...], 0.0)
            gi = jnp.dot(h1, wihr[...],
                         preferred_element_type=jnp.float32) + bihr[...]
            gh = jnp.dot(carry, whhr[...],
                         preferred_element_type=jnp.float32) + bhhr[...]
            r = jax.nn.sigmoid(gi[:, 0:h] + gh[:, 0:h])
            z = jax.nn.sigmoid(gi[:, h:2 * h] + gh[:, h:2 * h])
            nn = jnp.tanh(gi[:, 2 * h:3 * h] + r * gh[:, 2 * h:3 * h])
            carry = (1.0 - z) * nn + z * carry
            if emit_all:
                out[:, ti * h:(ti + 1) * h] = carry
        if not emit_all:
            out[...] = carry

    return pl.pallas_call(
        body,
        grid=(grid,),
        in_specs=[
            pl.BlockSpec((bn, fin), lambda i: (i, 0)),
            pl.BlockSpec((bn, fin), lambda i: (i, 0)),
            pl.BlockSpec((8, fout), lambda i: (0, 0)),
            pl.BlockSpec((8, fout), lambda i: (0, 0)),
            pl.BlockSpec((fin, fout), lambda i: (0, 0)),
            pl.BlockSpec((1, fout), lambda i: (0, 0)),
            pl.BlockSpec((1, fout), lambda i: (0, 0)),
            pl.BlockSpec((1, fout), lambda i: (0, 0)),
            pl.BlockSpec((h, h), lambda i: (0, 0)),
            pl.BlockSpec((1, h), lambda i: (0, 0)),
            pl.BlockSpec((h, 3 * h), lambda i: (0, 0)),
            pl.BlockSpec((h, 3 * h), lambda i: (0, 0)),
            pl.BlockSpec((1, 3 * h), lambda i: (0, 0)),
            pl.BlockSpec((1, 3 * h), lambda i: (0, 0)),
        ],
        out_specs=pl.BlockSpec((bn, out_w), lambda i: (i, 0)),
        out_shape=jax.ShapeDtypeStruct((n, out_w), jnp.float32),
    )(x, agg, sums, sqs, wbd, bbd, g_t, be_t, wb, bb,
      wih_t, whh_t, bih, bhh)


def _tc_head(hg, apt, wf1a, wf1b, bfc1, wfc2, bfc2, wfc3, bfc3):
    b, h = hg.shape
    h2 = wfc2.shape[1]

    def body(hgr, aptr, w1ar, w1br, b1r, w2r, b2r, w3r, b3r, out):
        y = (jnp.dot(hgr[...], w1ar[...], preferred_element_type=jnp.float32)
             + jnp.dot(aptr[...], w1br[...], preferred_element_type=jnp.float32)
             + b1r[...])
        y = jnp.where(y > 0, y, 0.1 * y)
        y = jnp.dot(y, w2r[...], preferred_element_type=jnp.float32) + b2r[...]
        y = jnp.where(y > 0, y, 0.05 * y)
        out[...] = (jnp.dot(y, w3r[...], preferred_element_type=jnp.float32)
                    + b3r[...])

    apt_w = apt.shape[1]
    h3 = wfc3.shape[0]
    return pl.pallas_call(
        body,
        out_shape=jax.ShapeDtypeStruct((b, 128), jnp.float32),
    )(hg, apt, wf1a, wf1b, bfc1.reshape(1, -1), wfc2,
      bfc2.reshape(1, -1), jnp.pad(wfc3, ((0, 0), (0, 127))),
      jnp.pad(bfc3, (0, 127)).reshape(1, -1))


def _block_diag(w, t):
    a, b = w.shape
    eye = jnp.eye(t, dtype=w.dtype)
    return (eye[:, None, :, None] * w[None, :, None, :]).reshape(t * a, t * b)


def kernel(x_seq, edge_index, node_idx, apart_feature, W1a, b1a, g1a, be1a,
           W1b, b1b, W2a, b2a, g2a, be2a, W2b, b2b, Wih, Whh, bih, bhh,
           Wfc1, bfc1, Wfc2, bfc2, Wfc3, bfc3):
    t, n, in_dim = x_seq.shape
    h = W1a.shape[1]
    chunk = 1536
    nchunk = _cdiv(n, chunk)
    if nchunk % NSC:
        nchunk += 1
    bn = 1000 if n % 1000 == 0 else n

    src, dst = edge_index[0], edge_index[1]
    psrc, pldst, ctrl = _prep_edges(src, dst, n, chunk, nchunk)

    f1 = t * in_dim
    f1p = _cdiv(f1, 128) * 128
    xp = jnp.pad(jnp.transpose(x_seq, (1, 0, 2)).reshape(n, f1),
                 ((0, 0), (0, f1p - f1)))

    w1a_bd = jnp.pad(_block_diag(W1a, t), ((0, f1p - f1), (0, 0)))
    b1a_t = jnp.tile(b1a, t).reshape(1, -1)
    g1a_t = jnp.tile(g1a, t).reshape(1, -1)
    be1a_t = jnp.tile(be1a, t).reshape(1, -1)
    w2a_bd = _block_diag(W2a, t)
    b2a_t = jnp.tile(b2a, t).reshape(1, -1)
    g2a_t = jnp.tile(g2a, t).reshape(1, -1)
    be2a_t = jnp.tile(be2a, t).reshape(1, -1)
    wih_t = Wih.T
    whh_t = Whh.T
    bih_r = bih.reshape(1, -1)
    bhh_r = bhh.reshape(1, -1)
    b1b_r = b1b.reshape(1, -1)
    b2b_r = b2b.reshape(1, -1)

    agg1 = _sc_aggregate(xp, psrc, pldst, ctrl, chunk, nchunk)[:n]
    s1, q1 = _tc_stats(xp, agg1, w1a_bd, b1a_t, bn, t)
    outs = _tc_main(xp, agg1, s1, q1, w1a_bd, b1a_t, g1a_t, be1a_t,
                    W1b, b1b_r, wih_t, whh_t, bih_r, bhh_r,
                    bn, t, h, emit_all=True)

    agg2 = _sc_aggregate(outs, psrc, pldst, ctrl, chunk, nchunk)[:n]
    s2, q2 = _tc_stats(outs, agg2, w2a_bd, b2a_t, bn, t)
    h_last = _tc_main(outs, agg2, s2, q2, w2a_bd, b2a_t, g2a_t, be2a_t,
                      W2b, b2b_r, wih_t, whh_t, bih_r, bhh_r,
                      bn, t, h, emit_all=False)

    hg = _sc_gather(jnp.pad(h_last, ((0, 0), (0, 128 - h))), node_idx)[:, :h]
    y = _tc_head(hg, apart_feature, Wfc1[:h], Wfc1[h:], bfc1,
                 Wfc2, bfc2, Wfc3, bfc3)
    return y[:, :1]


# speedup vs baseline: 37.8758x; 1.0933x over previous
import functools
import math

import jax
import jax.numpy as jnp
from jax import lax
from jax.experimental import pallas as pl
from jax.experimental.pallas import tpu as pltpu
from jax.experimental.pallas import tpu_sc as plsc

KB = 64
NSC = 2
NSUB = 16
NW = NSC * NSUB


def _cdiv(a, b):
    return (a + b - 1) // b


def _prep_edges(src, dst, n_nodes, chunk, nchunk):
    e = src.shape[0]
    ep_cap = _cdiv(e + nchunk * KB * 8 + (256 + SB) * KB, KB) * KB
    perm = jnp.argsort(dst)
    sdst = dst[perm]
    ssrc = src[perm]
    chunk_of = jnp.minimum(sdst // chunk, nchunk - 1)
    edges = jnp.arange(1, nchunk + 1, dtype=jnp.int32) * chunk
    bounds = jnp.searchsorted(sdst, edges, side="left").astype(jnp.int32)
    bounds = bounds.at[-1].set(e)
    starts = jnp.concatenate([jnp.zeros((1,), jnp.int32), bounds[:-1]])
    cnt = bounds - starts
    nb = ((_cdiv_arr(cnt) + 7) // 8) * 8
    pstart = jnp.concatenate([jnp.zeros((1,), jnp.int32),
                              jnp.cumsum(nb).astype(jnp.int32)])
    p = jnp.arange(ep_cap, dtype=jnp.int32)
    nbat = ep_cap // KB
    cid_b = jnp.clip(
        jnp.searchsorted(pstart, jnp.arange(nbat, dtype=jnp.int32),
                         side="right").astype(jnp.int32) - 1, 0, nchunk - 1)
    cid = jnp.repeat(cid_b, KB)
    local = p - pstart[cid] * KB
    eidx = starts[cid] + local
    valid = (local >= 0) & (local < cnt[cid])
    eidx_c = jnp.clip(eidx, 0, e - 1)
    psrc = jnp.where(valid, ssrc[eidx_c], (p * 997) % n_nodes)
    pldst = jnp.where(valid, sdst[eidx_c] - cid * chunk,
                      chunk + (p % NSUB))
    ctrl = (jnp.zeros((nchunk, 16), jnp.int32)
            .at[:, 0].set(pstart[:-1]).at[:, 1].set(nb).reshape(-1))
    return psrc, pldst, ctrl


def _cdiv_arr(x):
    return (x + (KB - 1)) // KB


SB = 32


def _sc_aggregate(x_rows, psrc, pldst, ctrl, chunk, nchunk):
    n, f = x_rows.shape
    g_cnt = f // 128
    npad = nchunk * chunk
    acc_rows = chunk + 128
    nzr = acc_rows // NSUB
    nwr = chunk // NSUB
    ctrl_len = ctrl.shape[0]
    nbat_cap = psrc.shape[0] // KB
    psrc2 = psrc.reshape(nbat_cap, KB)
    pldst2 = pldst.reshape(nbat_cap, KB)
    mesh = plsc.VectorSubcoreMesh(core_axis_name="c", subcore_axis_name="s")

    @functools.partial(
        pl.kernel, mesh=mesh,
        out_type=jax.ShapeDtypeStruct((npad, f), jnp.float32),
        scratch_types=(
            [pltpu.VMEM((SB, KB), jnp.int32),
             pltpu.VMEM((SB, KB), jnp.int32),
             pltpu.VMEM((NSUB, 128), jnp.float32),
             pltpu.VMEM((ctrl_len,), jnp.int32)]
            + [pltpu.VMEM((KB, 128), jnp.float32) for _ in range(2 * g_cnt)]
            + [pltpu.VMEM_SHARED((acc_rows, 128), jnp.float32)
               for _ in range(g_cnt)]
            + [pltpu.SemaphoreType.DMA, pltpu.SemaphoreType.DMA]
        ),
    )
    def agg_kernel(x_hbm, psrc_hbm, pldst_hbm, ctrl_hbm, out_hbm,
                   sidx_st, didx_st, zero_v, ctrl_v, *rest):
        rows_a = rest[:g_cnt]
        rows_bb = rest[g_cnt:2 * g_cnt]
        accs = rest[2 * g_cnt:3 * g_cnt]
        sem_a = rest[3 * g_cnt]
        sem_b = rest[3 * g_cnt + 1]
        c = lax.axis_index("c")
        s = lax.axis_index("s")
        pltpu.sync_copy(ctrl_hbm, ctrl_v)

        def zinit(i, _):
            zero_v[i // 8, pl.ds((i % 8) * 16, 16)] = jnp.zeros((16,),
                                                                jnp.float32)
            return 0
        lax.fori_loop(0, NSUB * 8, zinit, 0)

        def chunk_body(ci, _):
            chunk_id = ci * NSC + c
            rec = ctrl_v[pl.ds(chunk_id * 16, 16)]
            base_batch = rec[0]
            nb = rec[1]
            z0 = s * nzr
            for acc in accs:
                for k in range(nzr // NSUB):
                    pltpu.sync_copy(zero_v,
                                    acc.at[pl.ds(z0 + k * NSUB, NSUB)])
                rem = nzr % NSUB
                if rem:
                    pltpu.sync_copy(
                        zero_v.at[pl.ds(0, rem)],
                        acc.at[pl.ds(z0 + (nzr // NSUB) * NSUB, rem)])
            plsc.subcore_barrier()
            qs = (((nb + NSUB - 1) // NSUB + 7) // 8) * 8
            b0 = base_batch + s * qs
            myb = jnp.clip(nb - s * qs, 0, qs)

            def super_body(sb, _):
                off = pl.multiple_of(b0 + sb * SB, 8)
                pltpu.sync_copy(psrc_hbm.at[pl.ds(off, SB)], sidx_st)
                pltpu.sync_copy(pldst_hbm.at[pl.ds(off, SB)], didx_st)
                jb = jnp.minimum(SB, myb - sb * SB)

                def gsrc(j, g):
                    if g_cnt == 1:
                        return x_hbm.at[sidx_st.at[j]]
                    return x_hbm.at[sidx_st.at[j], pl.ds(g * 128, 128)]

                def fire(j, bufs, sem):
                    for g in range(g_cnt):
                        pltpu.async_copy(gsrc(j, g), bufs[g], sem)

                def drain_scatter(j, bufs, sem):
                    for g in range(g_cnt):
                        pltpu.make_async_copy(gsrc(j, g), bufs[g],
                                              sem).wait()
                        pltpu.sync_copy(bufs[g], accs[g].at[didx_st.at[j]],
                                        add=True)

                fire(0, rows_a, sem_a)

                def pair_body(q, _):
                    j0 = 2 * q
                    j1 = j0 + 1

                    @pl.when(j1 < jb)
                    def _():
                        fire(j1, rows_bb, sem_b)
                    drain_scatter(j0, rows_a, sem_a)

                    @pl.when(j0 + 2 < jb)
                    def _():
                        fire(j0 + 2, rows_a, sem_a)

                    @pl.when(j1 < jb)
                    def _():
                        drain_scatter(j1, rows_bb, sem_b)
                    return 0
                lax.fori_loop(0, (jb + 1) // 2, pair_body, 0)
                return 0
            lax.fori_loop(0, (myb + SB - 1) // SB, super_body, 0)
            plsc.subcore_barrier()
            w0 = s * nwr
            for g in range(g_cnt):
                if g_cnt == 1:
                    pltpu.sync_copy(
                        accs[g].at[pl.ds(w0, nwr)],
                        out_hbm.at[pl.ds(chunk_id * chunk + w0, nwr)])
                else:
                    pltpu.sync_copy(
                        accs[g].at[pl.ds(w0, nwr)],
                        out_hbm.at[pl.ds(chunk_id * chunk + w0, nwr),
                                   pl.ds(g * 128, 128)])
            plsc.subcore_barrier()
            return 0
        lax.fori_loop(0, nchunk // NSC, chunk_body, 0)

    return agg_kernel(x_rows, psrc2, pldst2, ctrl)


def _sc_gather(table, idx):
    n, f = table.shape
    b = idx.shape[0]
    per_w = b // (NSC * NSUB)
    mesh = plsc.VectorSubcoreMesh(core_axis_name="c", subcore_axis_name="s")

    @functools.partial(
        pl.kernel, mesh=mesh,
        out_type=jax.ShapeDtypeStruct((b, f), jnp.float32),
        scratch_types=[
            pltpu.VMEM((per_w,), jnp.int32),
            pltpu.VMEM((per_w, f), jnp.float32),
            pltpu.SemaphoreType.DMA,
        ],
    )
    def gather_kernel(tab_hbm, idx_hbm, out_hbm, idx_v, rows_v, sem):
        wid = lax.axis_index("s") * NSC + lax.axis_index("c")
        base = wid * per_w
        pltpu.sync_copy(idx_hbm.at[pl.ds(base, per_w)], idx_v)
        pltpu.async_copy(tab_hbm.at[idx_v], rows_v, sem).wait()
        pltpu.sync_copy(rows_v, out_hbm.at[pl.ds(base, per_w)])

    return gather_kernel(table, idx)


def _tc_stats(x, agg, wbd, bbd, bn, t):
    n, fin = x.shape
    fout = wbd.shape[1]
    grid = n // bn

    def body(xr, ar, wr, br, sums, sqs):
        i = pl.program_id(0)
        h = jnp.maximum(jnp.dot(xr[...] + ar[...], wr[...],
                                preferred_element_type=jnp.float32)
                        + br[...], 0.0)

        @pl.when(i == 0)
        def _():
            sums[...] = jnp.zeros((8, fout), jnp.float32)
            sqs[...] = jnp.zeros((8, fout), jnp.float32)
        r = i % 8
        part = jnp.sum(h, axis=0, keepdims=True)
        part2 = jnp.sum(h * h, axis=0, keepdims=True)
        sums[pl.ds(r, 1), :] += part
        sqs[pl.ds(r, 1), :] += part2

    return pl.pallas_call(
        body,
        grid=(grid,),
        in_specs=[
            pl.BlockSpec((bn, fin), lambda i: (i, 0)),
            pl.BlockSpec((bn, fin), lambda i: (i, 0)),
            pl.BlockSpec((fin, fout), lambda i: (0, 0)),
            pl.BlockSpec((1, fout), lambda i: (0, 0)),
        ],
        out_specs=[
            pl.BlockSpec((8, fout), lambda i: (0, 0)),
            pl.BlockSpec((8, fout), lambda i: (0, 0)),
        ],
        out_shape=[
            jax.ShapeDtypeStruct((8, fout), jnp.float32),
            jax.ShapeDtypeStruct((8, fout), jnp.float32),
        ],
    )(x, agg, wbd, bbd)


def _tc_main(x, agg, sums, sqs, wbd, bbd, g_t, be_t, wb, bb,
             wih_t, whh_t, bih, bhh, bn, t, h, emit_all):
    n, fin = x.shape
    fout = wbd.shape[1]
    n_total = float(n)
    grid = n // bn
    out_w = fout if emit_all else h

    def body(xr, ar, sums_r, sqs_r, wr, br, gr, ber, wbr, bbr,
             wihr, whhr, bihr, bhhr, out):
        hpre = jnp.maximum(jnp.dot(xr[...] + ar[...], wr[...],
                                   preferred_element_type=jnp.float32)
                           + br[...], 0.0)
        tot = jnp.sum(sums_r[...], axis=0, keepdims=True)
        tot2 = jnp.sum(sqs_r[...], axis=0, keepdims=True)
        mean = tot / n_total
        var = tot2 / n_total - mean * mean
        rstd = lax.rsqrt(var + 1e-5)
        scale = rstd * gr[...]
        shift = ber[...] - mean * scale
        hn = hpre * scale + shift
        carry = jnp.zeros((bn, h), jnp.float32)
        for ti in range(t):
            ht = hn[:, ti * h:(ti + 1) * h]
            h1 = jnp.maximum(jnp.dot(ht, wbr[...],
                                     preferred_element_type=jnp.float32)
                             + bbr[...], 0.0)
            gi = jnp.dot(h1, wihr[...],
                         preferred_element_type=jnp.float32) + bihr[...]
            gh = jnp.dot(carry, whhr[...],
                         preferred_element_type=jnp.float32) + bhhr[...]
            r = jax.nn.sigmoid(gi[:, 0:h] + gh[:, 0:h])
            z = jax.nn.sigmoid(gi[:, h:2 * h] + gh[:, h:2 * h])
            nn = jnp.tanh(gi[:, 2 * h:3 * h] + r * gh[:, 2 * h:3 * h])
            carry = (1.0 - z) * nn + z * carry
            if emit_all:
                out[:, ti * h:(ti + 1) * h] = carry
        if not emit_all:
            out[...] = carry

    return pl.pallas_call(
        body,
        grid=(grid,),
        in_specs=[
            pl.BlockSpec((bn, fin), lambda i: (i, 0)),
            pl.BlockSpec((bn, fin), lambda i: (i, 0)),
            pl.BlockSpec((8, fout), lambda i: (0, 0)),
            pl.BlockSpec((8, fout), lambda i: (0, 0)),
            pl.BlockSpec((fin, fout), lambda i: (0, 0)),
            pl.BlockSpec((1, fout), lambda i: (0, 0)),
            pl.BlockSpec((1, fout), lambda i: (0, 0)),
            pl.BlockSpec((1, fout), lambda i: (0, 0)),
            pl.BlockSpec((h, h), lambda i: (0, 0)),
            pl.BlockSpec((1, h), lambda i: (0, 0)),
            pl.BlockSpec((h, 3 * h), lambda i: (0, 0)),
            pl.BlockSpec((h, 3 * h), lambda i: (0, 0)),
            pl.BlockSpec((1, 3 * h), lambda i: (0, 0)),
            pl.BlockSpec((1, 3 * h), lambda i: (0, 0)),
        ],
        out_specs=pl.BlockSpec((bn, out_w), lambda i: (i, 0)),
        out_shape=jax.ShapeDtypeStruct((n, out_w), jnp.float32),
    )(x, agg, sums, sqs, wbd, bbd, g_t, be_t, wb, bb,
      wih_t, whh_t, bih, bhh)


def _tc_head(hg, apt, wf1a, wf1b, bfc1, wfc2, bfc2, wfc3, bfc3):
    b, h = hg.shape
    h2 = wfc2.shape[1]

    def body(hgr, aptr, w1ar, w1br, b1r, w2r, b2r, w3r, b3r, out):
        y = (jnp.dot(hgr[...], w1ar[...], preferred_element_type=jnp.float32)
             + jnp.dot(aptr[...], w1br[...], preferred_element_type=jnp.float32)
             + b1r[...])
        y = jnp.where(y > 0, y, 0.1 * y)
        y = jnp.dot(y, w2r[...], preferred_element_type=jnp.float32) + b2r[...]
        y = jnp.where(y > 0, y, 0.05 * y)
        out[...] = (jnp.dot(y, w3r[...], preferred_element_type=jnp.float32)
                    + b3r[...])

    apt_w = apt.shape[1]
    h3 = wfc3.shape[0]
    return pl.pallas_call(
        body,
        out_shape=jax.ShapeDtypeStruct((b, 128), jnp.float32),
    )(hg, apt, wf1a, wf1b, bfc1.reshape(1, -1), wfc2,
      bfc2.reshape(1, -1), jnp.pad(wfc3, ((0, 0), (0, 127))),
      jnp.pad(bfc3, (0, 127)).reshape(1, -1))


def _block_diag(w, t):
    a, b = w.shape
    eye = jnp.eye(t, dtype=w.dtype)
    return (eye[:, None, :, None] * w[None, :, None, :]).reshape(t * a, t * b)


def kernel(x_seq, edge_index, node_idx, apart_feature, W1a, b1a, g1a, be1a,
           W1b, b1b, W2a, b2a, g2a, be2a, W2b, b2b, Wih, Whh, bih, bhh,
           Wfc1, bfc1, Wfc2, bfc2, Wfc3, bfc3):
    t, n, in_dim = x_seq.shape
    h = W1a.shape[1]
    chunk = 1536
    nchunk = _cdiv(n, chunk)
    if nchunk % NSC:
        nchunk += 1
    bn = 1000 if n % 1000 == 0 else n

    src, dst = edge_index[0], edge_index[1]
    psrc, pldst, ctrl = _prep_edges(src, dst, n, chunk, nchunk)

    f1 = t * in_dim
    f1p = _cdiv(f1, 128) * 128
    xp = jnp.pad(jnp.transpose(x_seq, (1, 0, 2)).reshape(n, f1),
                 ((0, 0), (0, f1p - f1)))

    w1a_bd = jnp.pad(_block_diag(W1a, t), ((0, f1p - f1), (0, 0)))
    b1a_t = jnp.tile(b1a, t).reshape(1, -1)
    g1a_t = jnp.tile(g1a, t).reshape(1, -1)
    be1a_t = jnp.tile(be1a, t).reshape(1, -1)
    w2a_bd = _block_diag(W2a, t)
    b2a_t = jnp.tile(b2a, t).reshape(1, -1)
    g2a_t = jnp.tile(g2a, t).reshape(1, -1)
    be2a_t = jnp.tile(be2a, t).reshape(1, -1)
    wih_t = Wih.T
    whh_t = Whh.T
    bih_r = bih.reshape(1, -1)
    bhh_r = bhh.reshape(1, -1)
    b1b_r = b1b.reshape(1, -1)
    b2b_r = b2b.reshape(1, -1)

    agg1 = _sc_aggregate(xp, psrc, pldst, ctrl, chunk, nchunk)[:n]
    s1, q1 = _tc_stats(xp, agg1, w1a_bd, b1a_t, bn, t)
    outs = _tc_main(xp, agg1, s1, q1, w1a_bd, b1a_t, g1a_t, be1a_t,
                    W1b, b1b_r, wih_t, whh_t, bih_r, bhh_r,
                    bn, t, h, emit_all=True)

    agg2 = _sc_aggregate(outs, psrc, pldst, ctrl, chunk, nchunk)[:n]
    s2, q2 = _tc_stats(outs, agg2, w2a_bd, b2a_t, bn, t)
    h_last = _tc_main(outs, agg2, s2, q2, w2a_bd, b2a_t, g2a_t, be2a_t,
                      W2b, b2b_r, wih_t, whh_t, bih_r, bhh_r,
                      bn, t, h, emit_all=False)

    hg = _sc_gather(jnp.pad(h_last, ((0, 0), (0, 128 - h))), node_idx)[:, :h]
    y = _tc_head(hg, apart_feature, Wfc1[:h], Wfc1[h:], bfc1,
                 Wfc2, bfc2, Wfc3, bfc3)
    return y[:, :1]


# speedup vs baseline: 40.2887x; 1.0637x over previous
import functools
import math

import jax
import jax.numpy as jnp
from jax import lax
from jax.experimental import pallas as pl
from jax.experimental.pallas import tpu as pltpu
from jax.experimental.pallas import tpu_sc as plsc

KB = 64
NSC = 2
NSUB = 16
NW = NSC * NSUB


def _cdiv(a, b):
    return (a + b - 1) // b


def _prep_edges(src, dst, n_nodes, chunk, nchunk):
    e = src.shape[0]
    ep_cap = _cdiv(e + nchunk * KB * 8 + (256 + SB) * KB, KB) * KB
    sdst, ssrc = jax.lax.sort_key_val(dst, src, is_stable=False)
    chunk_of = jnp.minimum(sdst // chunk, nchunk - 1)
    edges = jnp.arange(1, nchunk + 1, dtype=jnp.int32) * chunk
    bounds = jnp.searchsorted(sdst, edges, side="left").astype(jnp.int32)
    bounds = bounds.at[-1].set(e)
    starts = jnp.concatenate([jnp.zeros((1,), jnp.int32), bounds[:-1]])
    cnt = bounds - starts
    nb = ((_cdiv_arr(cnt) + 7) // 8) * 8
    pstart = jnp.concatenate([jnp.zeros((1,), jnp.int32),
                              jnp.cumsum(nb).astype(jnp.int32)])
    p = jnp.arange(ep_cap, dtype=jnp.int32)
    nbat = ep_cap // KB
    cid_b = jnp.clip(
        jnp.searchsorted(pstart, jnp.arange(nbat, dtype=jnp.int32),
                         side="right").astype(jnp.int32) - 1, 0, nchunk - 1)
    cid = jnp.repeat(cid_b, KB)
    local = p - pstart[cid] * KB
    eidx = starts[cid] + local
    valid = (local >= 0) & (local < cnt[cid])
    eidx_c = jnp.clip(eidx, 0, e - 1)
    psrc = jnp.where(valid, ssrc[eidx_c], (p * 997) % n_nodes)
    pldst = jnp.where(valid, sdst[eidx_c] - cid * chunk,
                      chunk + (p % NSUB))
    ctrl = (jnp.zeros((nchunk, 16), jnp.int32)
            .at[:, 0].set(pstart[:-1]).at[:, 1].set(nb).reshape(-1))
    return psrc, pldst, ctrl


def _cdiv_arr(x):
    return (x + (KB - 1)) // KB


SB = 32


def _sc_aggregate(x_rows, psrc, pldst, ctrl, chunk, nchunk):
    n, f = x_rows.shape
    g_cnt = f // 128
    npad = nchunk * chunk
    acc_rows = chunk + 128
    nzr = acc_rows // NSUB
    nwr = chunk // NSUB
    ctrl_len = ctrl.shape[0]
    nbat_cap = psrc.shape[0] // KB
    psrc2 = psrc.reshape(nbat_cap, KB)
    pldst2 = pldst.reshape(nbat_cap, KB)
    mesh = plsc.VectorSubcoreMesh(core_axis_name="c", subcore_axis_name="s")

    @functools.partial(
        pl.kernel, mesh=mesh,
        out_type=jax.ShapeDtypeStruct((npad, f), jnp.float32),
        scratch_types=(
            [pltpu.VMEM((SB, KB), jnp.int32),
             pltpu.VMEM((SB, KB), jnp.int32),
             pltpu.VMEM((NSUB, 128), jnp.float32),
             pltpu.VMEM((ctrl_len,), jnp.int32)]
            + [pltpu.VMEM((KB, 128), jnp.float32) for _ in range(2 * g_cnt)]
            + [pltpu.VMEM_SHARED((acc_rows, 128), jnp.float32)
               for _ in range(g_cnt)]
            + [pltpu.SemaphoreType.DMA, pltpu.SemaphoreType.DMA]
        ),
    )
    def agg_kernel(x_hbm, psrc_hbm, pldst_hbm, ctrl_hbm, out_hbm,
                   sidx_st, didx_st, zero_v, ctrl_v, *rest):
        rows_a = rest[:g_cnt]
        rows_bb = rest[g_cnt:2 * g_cnt]
        accs = rest[2 * g_cnt:3 * g_cnt]
        sem_a = rest[3 * g_cnt]
        sem_b = rest[3 * g_cnt + 1]
        c = lax.axis_index("c")
        s = lax.axis_index("s")
        pltpu.sync_copy(ctrl_hbm, ctrl_v)

        def zinit(i, _):
            zero_v[i // 8, pl.ds((i % 8) * 16, 16)] = jnp.zeros((16,),
                                                                jnp.float32)
            return 0
        lax.fori_loop(0, NSUB * 8, zinit, 0)

        def chunk_body(ci, _):
            chunk_id = ci * NSC + c
            rec = ctrl_v[pl.ds(chunk_id * 16, 16)]
            base_batch = rec[0]
            nb = rec[1]
            z0 = s * nzr
            for acc in accs:
                for k in range(nzr // NSUB):
                    pltpu.sync_copy(zero_v,
                                    acc.at[pl.ds(z0 + k * NSUB, NSUB)])
                rem = nzr % NSUB
                if rem:
                    pltpu.sync_copy(
                        zero_v.at[pl.ds(0, rem)],
                        acc.at[pl.ds(z0 + (nzr // NSUB) * NSUB, rem)])
            plsc.subcore_barrier()
            qs = (((nb + NSUB - 1) // NSUB + 7) // 8) * 8
            b0 = base_batch + s * qs
            myb = jnp.clip(nb - s * qs, 0, qs)

            def super_body(sb, _):
                off = pl.multiple_of(b0 + sb * SB, 8)
                pltpu.sync_copy(psrc_hbm.at[pl.ds(off, SB)], sidx_st)
                pltpu.sync_copy(pldst_hbm.at[pl.ds(off, SB)], didx_st)
                jb = jnp.minimum(SB, myb - sb * SB)

                def gsrc(j, g):
                    if g_cnt == 1:
                        return x_hbm.at[sidx_st.at[j]]
                    return x_hbm.at[sidx_st.at[j], pl.ds(g * 128, 128)]

                def fire(j, bufs, sem):
                    for g in range(g_cnt):
                        pltpu.async_copy(gsrc(j, g), bufs[g], sem)

                def drain_scatter(j, bufs, sem):
                    for g in range(g_cnt):
                        pltpu.make_async_copy(gsrc(j, g), bufs[g],
                                              sem).wait()
                        pltpu.sync_copy(bufs[g], accs[g].at[didx_st.at[j]],
                                        add=True)

                fire(0, rows_a, sem_a)

                def pair_body(q, _):
                    j0 = 2 * q
                    j1 = j0 + 1

                    @pl.when(j1 < jb)
                    def _():
                        fire(j1, rows_bb, sem_b)
                    drain_scatter(j0, rows_a, sem_a)

                    @pl.when(j0 + 2 < jb)
                    def _():
                        fire(j0 + 2, rows_a, sem_a)

                    @pl.when(j1 < jb)
                    def _():
                        drain_scatter(j1, rows_bb, sem_b)
                    return 0
                lax.fori_loop(0, (jb + 1) // 2, pair_body, 0)
                return 0
            lax.fori_loop(0, (myb + SB - 1) // SB, super_body, 0)
            plsc.subcore_barrier()
            w0 = s * nwr
            for g in range(g_cnt):
                if g_cnt == 1:
                    pltpu.sync_copy(
                        accs[g].at[pl.ds(w0, nwr)],
                        out_hbm.at[pl.ds(chunk_id * chunk + w0, nwr)])
                else:
                    pltpu.sync_copy(
                        accs[g].at[pl.ds(w0, nwr)],
                        out_hbm.at[pl.ds(chunk_id * chunk + w0, nwr),
                                   pl.ds(g * 128, 128)])
            plsc.subcore_barrier()
            return 0
        lax.fori_loop(0, nchunk // NSC, chunk_body, 0)

    return agg_kernel(x_rows, psrc2, pldst2, ctrl)


def _sc_gather(table, idx):
    n, f = table.shape
    b = idx.shape[0]
    per_w = b // (NSC * NSUB)
    mesh = plsc.VectorSubcoreMesh(core_axis_name="c", subcore_axis_name="s")

    @functools.partial(
        pl.kernel, mesh=mesh,
        out_type=jax.ShapeDtypeStruct((b, f), jnp.float32),
        scratch_types=[
            pltpu.VMEM((per_w,), jnp.int32),
            pltpu.VMEM((per_w, f), jnp.float32),
            pltpu.SemaphoreType.DMA,
        ],
    )
    def gather_kernel(tab_hbm, idx_hbm, out_hbm, idx_v, rows_v, sem):
        wid = lax.axis_index("s") * NSC + lax.axis_index("c")
        base = wid * per_w
        pltpu.sync_copy(idx_hbm.at[pl.ds(base, per_w)], idx_v)
        pltpu.async_copy(tab_hbm.at[idx_v], rows_v, sem).wait()
        pltpu.sync_copy(rows_v, out_hbm.at[pl.ds(base, per_w)])

    return gather_kernel(table, idx)


def _tc_stats(x, agg, wbd, bbd, bn, t):
    n, fin = x.shape
    fout = wbd.shape[1]
    grid = n // bn

    def body(xr, ar, wr, br, sums, sqs):
        i = pl.program_id(0)
        h = jnp.maximum(jnp.dot(xr[...] + ar[...], wr[...],
                                preferred_element_type=jnp.float32)
                        + br[...], 0.0)

        @pl.when(i == 0)
        def _():
            sums[...] = jnp.zeros((8, fout), jnp.float32)
            sqs[...] = jnp.zeros((8, fout), jnp.float32)
        r = i % 8
        part = jnp.sum(h, axis=0, keepdims=True)
        part2 = jnp.sum(h * h, axis=0, keepdims=True)
        sums[pl.ds(r, 1), :] += part
        sqs[pl.ds(r, 1), :] += part2

    return pl.pallas_call(
        body,
        grid=(grid,),
        in_specs=[
            pl.BlockSpec((bn, fin), lambda i: (i, 0)),
            pl.BlockSpec((bn, fin), lambda i: (i, 0)),
            pl.BlockSpec((fin, fout), lambda i: (0, 0)),
            pl.BlockSpec((1, fout), lambda i: (0, 0)),
        ],
        out_specs=[
            pl.BlockSpec((8, fout), lambda i: (0, 0)),
            pl.BlockSpec((8, fout), lambda i: (0, 0)),
        ],
        out_shape=[
            jax.ShapeDtypeStruct((8, fout), jnp.float32),
            jax.ShapeDtypeStruct((8, fout), jnp.float32),
        ],
    )(x, agg, wbd, bbd)


def _tc_main(x, agg, sums, sqs, wbd, bbd, g_t, be_t, wb, bb,
             wih_t, whh_t, bih, bhh, bn, t, h, emit_all):
    n, fin = x.shape
    fout = wbd.shape[1]
    n_total = float(n)
    grid = n // bn
    out_w = fout if emit_all else h

    def body(xr, ar, sums_r, sqs_r, wr, br, gr, ber, wbr, bbr,
             wihr, whhr, bihr, bhhr, out):
        hpre = jnp.maximum(jnp.dot(xr[...] + ar[...], wr[...],
                                   preferred_element_type=jnp.float32)
                           + br[...], 0.0)
        tot = jnp.sum(sums_r[...], axis=0, keepdims=True)
        tot2 = jnp.sum(sqs_r[...], axis=0, keepdims=True)
        mean = tot / n_total
        var = tot2 / n_total - mean * mean
        rstd = lax.rsqrt(var + 1e-5)
        scale = rstd * gr[...]
        shift = ber[...] - mean * scale
        hn = hpre * scale + shift
        carry = jnp.zeros((bn, h), jnp.float32)
        for ti in range(t):
            ht = hn[:, ti * h:(ti + 1) * h]
            h1 = jnp.maximum(jnp.dot(ht, wbr[...],
                                     preferred_element_type=jnp.float32)
                             + bbr[...], 0.0)
            gi = jnp.dot(h1, wihr[...],
                         preferred_element_type=jnp.float32) + bihr[...]
            gh = jnp.dot(carry, whhr[...],
                         preferred_element_type=jnp.float32) + bhhr[...]
            r = jax.nn.sigmoid(gi[:, 0:h] + gh[:, 0:h])
            z = jax.nn.sigmoid(gi[:, h:2 * h] + gh[:, h:2 * h])
            nn = jnp.tanh(gi[:, 2 * h:3 * h] + r * gh[:, 2 * h:3 * h])
            carry = (1.0 - z) * nn + z * carry
            if emit_all:
                out[:, ti * h:(ti + 1) * h] = carry
        if not emit_all:
            out[...] = carry

    return pl.pallas_call(
        body,
        grid=(grid,),
        in_specs=[
            pl.BlockSpec((bn, fin), lambda i: (i, 0)),
            pl.BlockSpec((bn, fin), lambda i: (i, 0)),
            pl.BlockSpec((8, fout), lambda i: (0, 0)),
            pl.BlockSpec((8, fout), lambda i: (0, 0)),
            pl.BlockSpec((fin, fout), lambda i: (0, 0)),
            pl.BlockSpec((1, fout), lambda i: (0, 0)),
            pl.BlockSpec((1, fout), lambda i: (0, 0)),
            pl.BlockSpec((1, fout), lambda i: (0, 0)),
            pl.BlockSpec((h, h), lambda i: (0, 0)),
            pl.BlockSpec((1, h), lambda i: (0, 0)),
            pl.BlockSpec((h, 3 * h), lambda i: (0, 0)),
            pl.BlockSpec((h, 3 * h), lambda i: (0, 0)),
            pl.BlockSpec((1, 3 * h), lambda i: (0, 0)),
            pl.BlockSpec((1, 3 * h), lambda i: (0, 0)),
        ],
        out_specs=pl.BlockSpec((bn, out_w), lambda i: (i, 0)),
        out_shape=jax.ShapeDtypeStruct((n, out_w), jnp.float32),
    )(x, agg, sums, sqs, wbd, bbd, g_t, be_t, wb, bb,
      wih_t, whh_t, bih, bhh)


def _tc_head(hg, apt, wf1a, wf1b, bfc1, wfc2, bfc2, wfc3, bfc3):
    b, h = hg.shape
    h2 = wfc2.shape[1]

    def body(hgr, aptr, w1ar, w1br, b1r, w2r, b2r, w3r, b3r, out):
        y = (jnp.dot(hgr[...], w1ar[...], preferred_element_type=jnp.float32)
             + jnp.dot(aptr[...], w1br[...], preferred_element_type=jnp.float32)
             + b1r[...])
        y = jnp.where(y > 0, y, 0.1 * y)
        y = jnp.dot(y, w2r[...], preferred_element_type=jnp.float32) + b2r[...]
        y = jnp.where(y > 0, y, 0.05 * y)
        out[...] = (jnp.dot(y, w3r[...], preferred_element_type=jnp.float32)
                    + b3r[...])

    apt_w = apt.shape[1]
    h3 = wfc3.shape[0]
    return pl.pallas_call(
        body,
        out_shape=jax.ShapeDtypeStruct((b, 128), jnp.float32),
    )(hg, apt, wf1a, wf1b, bfc1.reshape(1, -1), wfc2,
      bfc2.reshape(1, -1), jnp.pad(wfc3, ((0, 0), (0, 127))),
      jnp.pad(bfc3, (0, 127)).reshape(1, -1))


def _block_diag(w, t):
    a, b = w.shape
    eye = jnp.eye(t, dtype=w.dtype)
    return (eye[:, None, :, None] * w[None, :, None, :]).reshape(t * a, t * b)


def kernel(x_seq, edge_index, node_idx, apart_feature, W1a, b1a, g1a, be1a,
           W1b, b1b, W2a, b2a, g2a, be2a, W2b, b2b, Wih, Whh, bih, bhh,
           Wfc1, bfc1, Wfc2, bfc2, Wfc3, bfc3):
    t, n, in_dim = x_seq.shape
    h = W1a.shape[1]
    chunk = 1536
    nchunk = _cdiv(n, chunk)
    if nchunk % NSC:
        nchunk += 1
    bn = 1000 if n % 1000 == 0 else n

    src, dst = edge_index[0], edge_index[1]
    psrc, pldst, ctrl = _prep_edges(src, dst, n, chunk, nchunk)

    f1 = t * in_dim
    f1p = _cdiv(f1, 128) * 128
    xp = jnp.pad(jnp.transpose(x_seq, (1, 0, 2)).reshape(n, f1),
                 ((0, 0), (0, f1p - f1)))

    w1a_bd = jnp.pad(_block_diag(W1a, t), ((0, f1p - f1), (0, 0)))
    b1a_t = jnp.tile(b1a, t).reshape(1, -1)
    g1a_t = jnp.tile(g1a, t).reshape(1, -1)
    be1a_t = jnp.tile(be1a, t).reshape(1, -1)
    w2a_bd = _block_diag(W2a, t)
    b2a_t = jnp.tile(b2a, t).reshape(1, -1)
    g2a_t = jnp.tile(g2a, t).reshape(1, -1)
    be2a_t = jnp.tile(be2a, t).reshape(1, -1)
    wih_t = Wih.T
    whh_t = Whh.T
    bih_r = bih.reshape(1, -1)
    bhh_r = bhh.reshape(1, -1)
    b1b_r = b1b.reshape(1, -1)
    b2b_r = b2b.reshape(1, -1)

    agg1 = _sc_aggregate(xp, psrc, pldst, ctrl, chunk, nchunk)[:n]
    s1, q1 = _tc_stats(xp, agg1, w1a_bd, b1a_t, bn, t)
    outs = _tc_main(xp, agg1, s1, q1, w1a_bd, b1a_t, g1a_t, be1a_t,
                    W1b, b1b_r, wih_t, whh_t, bih_r, bhh_r,
                    bn, t, h, emit_all=True)

    agg2 = _sc_aggregate(outs, psrc, pldst, ctrl, chunk, nchunk)[:n]
    s2, q2 = _tc_stats(outs, agg2, w2a_bd, b2a_t, bn, t)
    h_last = _tc_main(outs, agg2, s2, q2, w2a_bd, b2a_t, g2a_t, be2a_t,
                      W2b, b2b_r, wih_t, whh_t, bih_r, bhh_r,
                      bn, t, h, emit_all=False)

    hg = _sc_gather(jnp.pad(h_last, ((0, 0), (0, 128 - h))), node_idx)[:, :h]
    y = _tc_head(hg, apart_feature, Wfc1[:h], Wfc1[h:], bfc1,
                 Wfc2, bfc2, Wfc3, bfc3)
    return y[:, :1]
